# Initial kernel scaffold; baseline (speedup 1.0000x reference)
import jax
import jax.numpy as jnp
from jax.experimental import pallas as pl


def kernel(x, edge_index, batch, W_init, b_init, W_head, b_head, W_body, b_body, W_tail, b_tail, W1, b1, gamma, beta, W2, b2):
    raise NotImplementedError("write your pallas kernel here")


# speedup vs baseline: 3.2169x; 3.2169x over previous
import functools

import jax
import jax.numpy as jnp
from jax import lax
from jax.experimental import pallas as pl
from jax.experimental.pallas import tpu as pltpu
from jax.experimental.pallas import tpu_sc as plsc

N = 10000
E = 320000
NC = 2
NS = 16
NW = NC * NS
NLOC = 320
NPAD = NW * NLOC
EPAD = E + 4096
FLUSH = 2048
BUF = FLUSH + 16
F_CH = 6400
F_NCH = E // F_CH
DUMMY_ROW = NLOC
ACC_ROWS = NLOC + 16
DUMMY_PK = DUMMY_ROW << 14
ALPHA = 0.01
EPS = 1e-5
NEG_INF = float("-inf")

_MESH = dict(core_axis_name="c", subcore_axis_name="s", num_cores=NC,
             num_subcores=NS)


def _wid():
    return lax.axis_index("s") * NC + lax.axis_index("c")


def _mo8(v):
    return pl.multiple_of(v, 8)


def _bucket_body(src_hbm, dst_hbm, packed_hbm, counts_hbm, deg_hbm,
                 src_v, dst_v, buf, cnt16, degacc, pkv, sem):
    wid = _wid()
    lo = wid * NLOC
    pbase = wid * EPAD

    def inner(j, carry):
        cnt, goff = carry
        sv = src_v[pl.ds(j * 16, 16)]
        dv = dst_v[pl.ds(j * 16, 16)]
        m = (dv >= lo) & (dv < lo + NLOC)
        pk = ((dv - lo) << 14) | sv
        incl = plsc.cumsum(m.astype(jnp.int32))
        plsc.store_scatter(buf, [cnt + incl - 1], pk, mask=m)
        cnt = cnt + incl[15]

        def flush(args):
            c, g = args
            pltpu.sync_copy(buf.at[pl.ds(0, FLUSH)],
                            packed_hbm.at[pl.ds(_mo8(pbase + g), FLUSH)])
            tail = buf[pl.ds(FLUSH, 16)]
            buf[pl.ds(0, 16)] = tail
            return c - FLUSH, g + FLUSH

        return lax.cond(cnt >= FLUSH, flush, lambda a: a, (cnt, goff))

    def outer(c, carry):
        off = c * F_CH
        pltpu.sync_copy(src_hbm.at[pl.ds(_mo8(off), F_CH)], src_v)
        pltpu.sync_copy(dst_hbm.at[pl.ds(_mo8(off), F_CH)], dst_v)
        return lax.fori_loop(0, F_CH // 16, inner, carry)

    cnt, goff = lax.fori_loop(0, F_NCH, outer,
                              (jnp.int32(0), jnp.int32(0)))
    pltpu.sync_copy(buf.at[pl.ds(0, FLUSH)],
                    packed_hbm.at[pl.ds(_mo8(pbase + goff), FLUSH)])
    cnt = goff + cnt
    cnt16[...] = jnp.full((16,), cnt, jnp.int32)
    pltpu.sync_copy(cnt16, counts_hbm.at[wid])

    zeros = jnp.zeros((16,), jnp.float32)

    def zbody(i, _):
        degacc[pl.ds(i * 16, 16)] = zeros
        return 0

    lax.fori_loop(0, ACC_ROWS, zbody, 0)
    ones = jnp.ones((16,), jnp.float32)

    def degbody(c, _):
        pltpu.sync_copy(packed_hbm.at[pl.ds(_mo8(pbase + c * 128), 128)], pkv)
        for j in range(8):
            pos = c * 128 + j * 16 + lax.iota(jnp.int32, 16)
            v = pkv[pl.ds(j * 16, 16)]
            v = jnp.where(pos < cnt, v, jnp.int32(DUMMY_PK))
            dloc = v >> 14
            for l in range(16):
                dl = dloc[l]
                degacc[pl.ds(dl * 16, 16)] += ones
        return 0

    lax.fori_loop(0, (cnt + 127) // 128, degbody, 0)
    pltpu.sync_copy(degacc.at[pl.ds(0, NLOC * 16)],
                    deg_hbm.at[pl.ds(_mo8(wid * NLOC * 16), NLOC * 16)])


def _make_bucket():
    return pl.kernel(
        _bucket_body,
        out_type=(
            jax.ShapeDtypeStruct((NW * EPAD,), jnp.int32),
            jax.ShapeDtypeStruct((NW, 16), jnp.int32),
            jax.ShapeDtypeStruct((NPAD * 16,), jnp.float32),
        ),
        mesh=plsc.VectorSubcoreMesh(**_MESH),
        compiler_params=pltpu.CompilerParams(needs_layout_passes=False, use_tc_tiling_on_sc=False),
        scratch_types=[
            pltpu.VMEM((F_CH,), jnp.int32),
            pltpu.VMEM((F_CH,), jnp.int32),
            pltpu.VMEM((BUF,), jnp.int32),
            pltpu.VMEM((16,), jnp.int32),
            pltpu.VMEM((ACC_ROWS * 16,), jnp.float32),
            pltpu.VMEM((128,), jnp.int32),
            pltpu.SemaphoreType.DMA,
        ],
    )


def _prop_body(D, CH, y_hbm, packed_hbm, counts_hbm, out_hbm,
               pkv, sidx, rows, acc, cntv, sem):
    wid = _wid()
    lo = wid * NLOC
    pbase = wid * EPAD
    pltpu.sync_copy(counts_hbm.at[wid], cntv)
    cnt = cntv[pl.ds(0, 16)][0]
    zeros = jnp.zeros((16,), jnp.float32)
    ng = D // 16

    def zbody(r, _):
        for g in range(ng):
            acc[r, pl.ds(g * 16, 16)] = zeros
        return 0

    lax.fori_loop(0, ACC_ROWS, zbody, 0)

    def chunk(c, _):
        cbase = c * CH
        pltpu.sync_copy(packed_hbm.at[pl.ds(_mo8(pbase + cbase), CH)], pkv)
        for j in range(CH // 16):
            pos = cbase + j * 16 + lax.iota(jnp.int32, 16)
            v = pkv[pl.ds(j * 16, 16)]
            v = jnp.where(pos < cnt, v, jnp.int32(DUMMY_PK))
            pkv[pl.ds(j * 16, 16)] = v
            sidx[pl.ds(j * 16, 16)] = v & 0x3FFF
        pltpu.async_copy(y_hbm.at[sidx], rows, sem).wait()
        for j in range(CH // 16):
            dloc = pkv[pl.ds(j * 16, 16)] >> 14
            for l in range(16):
                e = j * 16 + l
                dl = dloc[l]
                for g in range(ng):
                    acc[dl, pl.ds(g * 16, 16)] += rows[e, pl.ds(g * 16, 16)]
        return 0

    lax.fori_loop(0, (cnt + CH - 1) // CH, chunk, 0)
    pltpu.sync_copy(acc.at[pl.ds(0, NLOC)], out_hbm.at[pl.ds(_mo8(lo), NLOC)])


def _make_prop(D):
    CH = 8192 // D
    return pl.kernel(
        functools.partial(_prop_body, D, CH),
        out_type=jax.ShapeDtypeStruct((NPAD, D), jnp.float32),
        mesh=plsc.VectorSubcoreMesh(**_MESH),
        compiler_params=pltpu.CompilerParams(needs_layout_passes=False, use_tc_tiling_on_sc=False),
        scratch_types=[
            pltpu.VMEM((CH,), jnp.int32),
            pltpu.VMEM((CH,), jnp.int32),
            pltpu.VMEM((CH, D), jnp.float32),
            pltpu.VMEM((ACC_ROWS, D), jnp.float32),
            pltpu.VMEM((16,), jnp.int32),
            pltpu.SemaphoreType.DMA,
        ],
    )


def _pool_body(x4, x1, x2, x3, batch_hbm, out_hbm, bv, rowbuf, acc, sem):
    wid = _wid()
    lo = wid * NLOC
    nrows = jnp.minimum(jnp.int32(NLOC), jnp.int32(N) - lo)
    ninf = jnp.full((16,), NEG_INF, jnp.float32)

    def ibody(r, _):
        for g in range(16):
            acc[r, pl.ds(g * 16, 16)] = ninf
        return 0

    lax.fori_loop(0, 64, ibody, 0)
    pltpu.sync_copy(batch_hbm.at[pl.ds(_mo8(lo), NLOC)], bv)
    for ai, arr in enumerate((x4, x1, x2, x3)):
        pltpu.sync_copy(arr.at[pl.ds(_mo8(lo), NLOC)], rowbuf)

        def rbody(j, _):
            bvec = bv[pl.ds(j * 16, 16)]
            for l in range(16):
                g = bvec[l]
                r = j * 16 + l
                for fg in range(4):
                    col = ai * 64 + fg * 16
                    cur = acc[g, pl.ds(col, 16)]
                    acc[g, pl.ds(col, 16)] = jnp.maximum(
                        cur, rowbuf[r, pl.ds(fg * 16, 16)])
            return 0

        lax.fori_loop(0, nrows // 16, rbody, 0)
    pltpu.sync_copy(acc, out_hbm.at[wid])


def _make_pool():
    return pl.kernel(
        _pool_body,
        out_type=jax.ShapeDtypeStruct((NW, 64, 256), jnp.float32),
        mesh=plsc.VectorSubcoreMesh(**_MESH),
        compiler_params=pltpu.CompilerParams(needs_layout_passes=False, use_tc_tiling_on_sc=False),
        scratch_types=[
            pltpu.VMEM((NLOC,), jnp.int32),
            pltpu.VMEM((NLOC, 64), jnp.float32),
            pltpu.VMEM((64, 256), jnp.float32),
            pltpu.SemaphoreType.DMA,
        ],
    )


_bucket = _make_bucket()
_prop128 = _make_prop(128)
_prop64 = _make_prop(64)
_pool = _make_pool()

_BR = 256
_GRID = NPAD // _BR


def _blk(w):
    return pl.BlockSpec((_BR, w), lambda i: (i, 0))


def _full(shape):
    nd = len(shape)
    return pl.BlockSpec(shape, lambda i, _n=nd: (0,) * _n)


def _t0_body(deg_ref, x_ref, dis_ref, dis64_ref, u_ref):
    deg = deg_ref[:, 0:1]
    dis = jnp.where(deg > 0, lax.rsqrt(jnp.maximum(deg, 1e-12)), 0.0)
    dis64_ref[...] = jnp.broadcast_to(dis, (_BR, 64))
    dis = jnp.broadcast_to(dis, (_BR, 128))
    dis_ref[...] = dis
    u_ref[...] = dis * x_ref[...]


def _t0(deg16, xp):
    return pl.pallas_call(
        _t0_body,
        grid=(_GRID,),
        in_specs=[_blk(16), _blk(128)],
        out_specs=[_blk(128), _blk(64), _blk(128)],
        out_shape=[jax.ShapeDtypeStruct((NPAD, 128), jnp.float32),
                   jax.ShapeDtypeStruct((NPAD, 64), jnp.float32),
                   jax.ShapeDtypeStruct((NPAD, 128), jnp.float32)],
    )(deg16, xp)


def _t1_body(s1_ref, dis_ref, u_ref):
    dis = dis_ref[...]
    u_ref[...] = -(dis * dis) * s1_ref[...]


def _t1(S1, dis, D):
    return pl.pallas_call(
        _t1_body,
        grid=(_GRID,),
        in_specs=[_blk(D), _blk(D)],
        out_specs=_blk(D),
        out_shape=jax.ShapeDtypeStruct((NPAD, D), jnp.float32),
    )(S1, dis)


def _layer_math(h, S1, S2, dis, W, b):
    Tx1 = -dis * S1
    Tx2 = -2.0 * dis * S2 - h
    o = (jnp.dot(h, W[0], preferred_element_type=jnp.float32)
         + jnp.dot(Tx1, W[1], preferred_element_type=jnp.float32)
         + jnp.dot(Tx2, W[2], preferred_element_type=jnp.float32) + b)
    return jnp.where(o >= 0, o, ALPHA * o)


def _t2_body(emit_u, h_ref, s1_ref, s2_ref, dis_ref, w_ref, b_ref,
             out_ref, *rest):
    out = _layer_math(h_ref[...], s1_ref[...], s2_ref[...], dis_ref[...],
                      w_ref[...], b_ref[...])
    out_ref[...] = out
    if emit_u:
        rest[0][...] = dis_ref[:, 0:64] * out


def _t2(h, S1, S2, dis, W, b, D, emit_u):
    n_out = 2 if emit_u else 1
    outs = pl.pallas_call(
        functools.partial(_t2_body, emit_u),
        grid=(_GRID,),
        in_specs=[_blk(D), _blk(D), _blk(D), _blk(D),
                  _full((3, D, 64)), _full((1, 64))],
        out_specs=[_blk(64)] * n_out,
        out_shape=[jax.ShapeDtypeStruct((NPAD, 64), jnp.float32)] * n_out,
    )(h, S1, S2, dis, W, b)
    return outs if emit_u else outs[0]


def _t2_skip_body(h_ref, s1_ref, s2_ref, dis_ref, w_ref, b_ref, hs_ref,
                  x2_ref, x3_ref, u_ref):
    x2 = _layer_math(h_ref[...], s1_ref[...], s2_ref[...], dis_ref[...],
                     w_ref[...], b_ref[...])
    x3 = hs_ref[...] + x2
    x2_ref[...] = x2
    x3_ref[...] = x3
    u_ref[...] = dis_ref[...] * x3


def _t2_skip(h, S1, S2, dis, W, b, hskip):
    return pl.pallas_call(
        _t2_skip_body,
        grid=(_GRID,),
        in_specs=[_blk(64)] * 4 + [_full((3, 64, 64)), _full((1, 64)),
                                   _blk(64)],
        out_specs=[_blk(64)] * 3,
        out_shape=[jax.ShapeDtypeStruct((NPAD, 64), jnp.float32)] * 3,
    )(h, S1, S2, dis, W, b, hskip)


def _finish_body(part_ref, w1_ref, b1_ref, g_ref, be_ref, w2_ref, b2_ref,
                 out_ref):
    pooled = jnp.max(part_ref[...], axis=0)
    h = jnp.dot(pooled, w1_ref[...], preferred_element_type=jnp.float32)
    h = h + b1_ref[...]
    h = g_ref[...] * h * (1.0 / jnp.sqrt(1.0 + EPS)) + be_ref[...]
    h = jnp.maximum(h, 0.0)
    out_ref[...] = (jnp.dot(h, w2_ref[...],
                            preferred_element_type=jnp.float32) + b2_ref[...])


def _finish(part, W1, b1, gamma, beta, W2, b2):
    return pl.pallas_call(
        _finish_body,
        grid=(1,),
        in_specs=[_full((NW, 64, 256)), _full((256, 64)), _full((1, 64)),
                  _full((1, 64)), _full((1, 64)), _full((64, 10)),
                  _full((1, 10))],
        out_specs=_full((64, 10)),
        out_shape=jax.ShapeDtypeStruct((64, 10), jnp.float32),
    )(part, W1, b1, gamma, beta, W2, b2)


def kernel(x, edge_index, batch, W_init, b_init, W_head, b_head,
           W_body, b_body, W_tail, b_tail, W1, b1, gamma, beta, W2, b2):
    src = edge_index[0]
    dst = edge_index[1]
    xp = jnp.pad(x, ((0, NPAD - N), (0, 0)))
    batchp = jnp.pad(batch, (0, NPAD - N))

    packed, counts, deg_flat = _bucket(src, dst)
    deg16 = deg_flat.reshape(NPAD, 16)
    dis, dis64, u0 = _t0(deg16, xp)

    def cheb(h, u_in, W, b, D, prop, emit_u):
        d = dis if D == 128 else dis64
        S1 = prop(u_in, packed, counts)
        u1 = _t1(S1, d, D)
        S2 = prop(u1, packed, counts)
        return _t2(h, S1, S2, d, W, b.reshape(1, 64), D, emit_u), S1, S2

    (h0, u_h0), _, _ = cheb(xp, u0, W_init, b_init, 128, _prop128, True)
    (x1, u_x1), _, _ = cheb(h0, u_h0, W_head, b_head, 64, _prop64, True)
    S1 = _prop64(u_x1, packed, counts)
    u1 = _t1(S1, dis64, 64)
    S2 = _prop64(u1, packed, counts)
    x2, x3, u_x3 = _t2_skip(x1, S1, S2, dis64, W_body,
                            b_body.reshape(1, 64), h0)
    x4, _, _ = cheb(x3, u_x3, W_tail, b_tail, 64, _prop64, False)

    part = _pool(x4, x1, x2, x3, batchp)
    return _finish(part, W1, b1.reshape(1, 64), gamma.reshape(1, 64),
                   beta.reshape(1, 64), W2, b2.reshape(1, 10))


# speedup vs baseline: 3.6832x; 1.1450x over previous
import functools

import jax
import jax.numpy as jnp
from jax import lax
from jax.experimental import pallas as pl
from jax.experimental.pallas import tpu as pltpu
from jax.experimental.pallas import tpu_sc as plsc

N = 10000
E = 320000
NC = 2
NS = 16
NW = NC * NS
NLOC = 320
NPAD = NW * NLOC
EPAD = E + 4096
FLUSH = 2048
BUF = FLUSH + 16
F_CH = 6400
F_NCH = E // F_CH
DUMMY_ROW = NLOC
ACC_ROWS = NLOC + 16
DUMMY_PK = DUMMY_ROW << 14
ALPHA = 0.01
EPS = 1e-5
NEG_INF = float("-inf")

_MESH = dict(core_axis_name="c", subcore_axis_name="s", num_cores=NC,
             num_subcores=NS)


def _wid():
    return lax.axis_index("s") * NC + lax.axis_index("c")


def _mo8(v):
    return pl.multiple_of(v, 8)


def _bucket_body(src_hbm, dst_hbm, packed_hbm, counts_hbm, deg_hbm,
                 src_v0, dst_v0, src_v1, dst_v1, buf, cnt16, degacc, pkv,
                 semA, semB):
    wid = _wid()
    lo = wid * NLOC
    pbase = wid * EPAD

    def scan(src_v, dst_v, carry):
        def inner(j, carry):
            cnt, goff = carry
            sv = src_v[pl.ds(j * 16, 16)]
            dv = dst_v[pl.ds(j * 16, 16)]
            m = (dv >= lo) & (dv < lo + NLOC)
            pk = ((dv - lo) << 14) | sv
            incl = plsc.cumsum(m.astype(jnp.int32))
            plsc.store_scatter(buf, [cnt + incl - 1], pk, mask=m)
            cnt = cnt + incl[15]

            def flush(args):
                c, g = args
                pltpu.sync_copy(buf.at[pl.ds(0, FLUSH)],
                                packed_hbm.at[pl.ds(_mo8(pbase + g), FLUSH)])
                tail = buf[pl.ds(FLUSH, 16)]
                buf[pl.ds(0, 16)] = tail
                return c - FLUSH, g + FLUSH

            return lax.cond(cnt >= FLUSH, flush, lambda a: a, (cnt, goff))

        return lax.fori_loop(0, F_CH // 16, inner, carry)

    def start_load(c, src_v, dst_v, sem):
        off = _mo8(c * F_CH)
        pltpu.async_copy(src_hbm.at[pl.ds(off, F_CH)], src_v, sem)
        pltpu.async_copy(dst_hbm.at[pl.ds(off, F_CH)], dst_v, sem)

    def wait_load(src_v, dst_v, sem):
        pltpu.make_async_copy(src_hbm.at[pl.ds(0, F_CH)], src_v, sem).wait()
        pltpu.make_async_copy(dst_hbm.at[pl.ds(0, F_CH)], dst_v, sem).wait()

    start_load(0, src_v0, dst_v0, semA)

    def pair(t, carry):
        start_load(2 * t + 1, src_v1, dst_v1, semB)
        wait_load(src_v0, dst_v0, semA)
        carry = scan(src_v0, dst_v0, carry)
        start_load(jnp.minimum(2 * t + 2, F_NCH - 1), src_v0, dst_v0, semA)
        wait_load(src_v1, dst_v1, semB)
        return scan(src_v1, dst_v1, carry)

    cnt, goff = lax.fori_loop(0, F_NCH // 2, pair,
                              (jnp.int32(0), jnp.int32(0)))
    wait_load(src_v0, dst_v0, semA)
    pltpu.sync_copy(buf.at[pl.ds(0, FLUSH)],
                    packed_hbm.at[pl.ds(_mo8(pbase + goff), FLUSH)])
    cnt = goff + cnt
    cnt16[...] = jnp.full((16,), cnt, jnp.int32)
    pltpu.sync_copy(cnt16, counts_hbm.at[wid])

    zeros = jnp.zeros((16,), jnp.float32)

    def zbody(i, _):
        degacc[pl.ds(i * 16, 16)] = zeros
        return 0

    lax.fori_loop(0, ACC_ROWS, zbody, 0)
    ones = jnp.ones((16,), jnp.float32)

    def degbody(c, _):
        pltpu.sync_copy(packed_hbm.at[pl.ds(_mo8(pbase + c * 1024), 1024)],
                        pkv)

        def degvec(j, _):
            pos = c * 1024 + j * 16 + lax.iota(jnp.int32, 16)
            v = pkv[pl.ds(j * 16, 16)]
            v = jnp.where(pos < cnt, v, jnp.int32(DUMMY_PK))
            dloc = v >> 14
            for l in range(16):
                dl = dloc[l]
                plsc.addupdate(degacc.at[pl.ds(dl * 16, 16)], ones)
            return 0

        lax.fori_loop(0, 64, degvec, 0)
        return 0

    lax.fori_loop(0, (cnt + 1023) // 1024, degbody, 0)
    pltpu.sync_copy(degacc.at[pl.ds(0, NLOC * 16)],
                    deg_hbm.at[pl.ds(_mo8(wid * NLOC * 16), NLOC * 16)])


def _make_bucket():
    return pl.kernel(
        _bucket_body,
        out_type=(
            jax.ShapeDtypeStruct((NW * EPAD,), jnp.int32),
            jax.ShapeDtypeStruct((NW, 16), jnp.int32),
            jax.ShapeDtypeStruct((NPAD * 16,), jnp.float32),
        ),
        mesh=plsc.VectorSubcoreMesh(**_MESH),
        compiler_params=pltpu.CompilerParams(needs_layout_passes=False, use_tc_tiling_on_sc=False),
        scratch_types=[
            pltpu.VMEM((F_CH,), jnp.int32),
            pltpu.VMEM((F_CH,), jnp.int32),
            pltpu.VMEM((F_CH,), jnp.int32),
            pltpu.VMEM((F_CH,), jnp.int32),
            pltpu.VMEM((BUF,), jnp.int32),
            pltpu.VMEM((16,), jnp.int32),
            pltpu.VMEM((ACC_ROWS * 16,), jnp.float32),
            pltpu.VMEM((1024,), jnp.int32),
            pltpu.SemaphoreType.DMA,
            pltpu.SemaphoreType.DMA,
        ],
    )


def _prop_body(D, CH, y_hbm, packed_hbm, counts_hbm, out_hbm,
               pkv0, pkv1, sidx0, sidx1, rows0, rows1, acc, cntv,
               sem0, sem1):
    wid = _wid()
    lo = wid * NLOC
    pbase = wid * EPAD
    pltpu.sync_copy(counts_hbm.at[wid], cntv)
    cnt = cntv[pl.ds(0, 16)][0]
    zeros = jnp.zeros((16,), jnp.float32)
    ng = D // 16
    nv = CH // 16

    def zbody(r, _):
        for g in range(ng):
            acc[r, pl.ds(g * 16, 16)] = zeros
        return 0

    lax.fori_loop(0, ACC_ROWS, zbody, 0)

    def load_unpack(c, pkv, sidx):
        cbase = c * CH
        pltpu.sync_copy(packed_hbm.at[pl.ds(_mo8(pbase + cbase), CH)], pkv)
        for j in range(nv):
            pos = cbase + j * 16 + lax.iota(jnp.int32, 16)
            v = pkv[pl.ds(j * 16, 16)]
            v = jnp.where(pos < cnt, v, jnp.int32(DUMMY_PK))
            pkv[pl.ds(j * 16, 16)] = v
            sidx[pl.ds(j * 16, 16)] = v & 0x3FFF

    def accumulate(pkv, rows):
        for j in range(nv):
            dloc = pkv[pl.ds(j * 16, 16)] >> 14
            for l in range(16):
                e = j * 16 + l
                dl = dloc[l]
                for g in range(ng):
                    plsc.addupdate(acc.at[dl, pl.ds(g * 16, 16)],
                                   rows[e, pl.ds(g * 16, 16)])

    nch = (cnt + CH - 1) // CH
    load_unpack(0, pkv0, sidx0)
    pltpu.async_copy(y_hbm.at[sidx0], rows0, sem0)

    def pair(t, _):
        load_unpack(2 * t + 1, pkv1, sidx1)
        pltpu.async_copy(y_hbm.at[sidx1], rows1, sem1)
        pltpu.make_async_copy(y_hbm.at[sidx0], rows0, sem0).wait()
        accumulate(pkv0, rows0)
        load_unpack(2 * t + 2, pkv0, sidx0)
        pltpu.async_copy(y_hbm.at[sidx0], rows0, sem0)
        pltpu.make_async_copy(y_hbm.at[sidx1], rows1, sem1).wait()
        accumulate(pkv1, rows1)
        return 0

    lax.fori_loop(0, (nch + 1) // 2, pair, 0)
    pltpu.make_async_copy(y_hbm.at[sidx0], rows0, sem0).wait()
    pltpu.sync_copy(acc.at[pl.ds(0, NLOC)], out_hbm.at[pl.ds(_mo8(lo), NLOC)])


def _make_prop(D):
    CH = 8192 // D
    return pl.kernel(
        functools.partial(_prop_body, D, CH),
        out_type=jax.ShapeDtypeStruct((NPAD, D), jnp.float32),
        mesh=plsc.VectorSubcoreMesh(**_MESH),
        compiler_params=pltpu.CompilerParams(needs_layout_passes=False, use_tc_tiling_on_sc=False),
        scratch_types=[
            pltpu.VMEM((CH,), jnp.int32),
            pltpu.VMEM((CH,), jnp.int32),
            pltpu.VMEM((CH,), jnp.int32),
            pltpu.VMEM((CH,), jnp.int32),
            pltpu.VMEM((CH, D), jnp.float32),
            pltpu.VMEM((CH, D), jnp.float32),
            pltpu.VMEM((ACC_ROWS, D), jnp.float32),
            pltpu.VMEM((16,), jnp.int32),
            pltpu.SemaphoreType.DMA,
            pltpu.SemaphoreType.DMA,
        ],
    )


def _pool_body(x4, x1, x2, x3, batch_hbm, out_hbm, bv, rowbuf, acc, sem):
    wid = _wid()
    lo = wid * NLOC
    nrows = jnp.minimum(jnp.int32(NLOC), jnp.int32(N) - lo)
    ninf = jnp.full((16,), NEG_INF, jnp.float32)

    def ibody(r, _):
        for g in range(16):
            acc[r, pl.ds(g * 16, 16)] = ninf
        return 0

    lax.fori_loop(0, 64, ibody, 0)
    pltpu.sync_copy(batch_hbm.at[pl.ds(_mo8(lo), NLOC)], bv)
    for ai, arr in enumerate((x4, x1, x2, x3)):
        pltpu.sync_copy(arr.at[pl.ds(_mo8(lo), NLOC)], rowbuf)

        def rbody(j, _):
            bvec = bv[pl.ds(j * 16, 16)]
            for l in range(16):
                g = bvec[l]
                r = j * 16 + l
                for fg in range(4):
                    col = ai * 64 + fg * 16
                    cur = acc[g, pl.ds(col, 16)]
                    acc[g, pl.ds(col, 16)] = jnp.maximum(
                        cur, rowbuf[r, pl.ds(fg * 16, 16)])
            return 0

        lax.fori_loop(0, nrows // 16, rbody, 0)
    pltpu.sync_copy(acc, out_hbm.at[wid])


def _make_pool():
    return pl.kernel(
        _pool_body,
        out_type=jax.ShapeDtypeStruct((NW, 64, 256), jnp.float32),
        mesh=plsc.VectorSubcoreMesh(**_MESH),
        compiler_params=pltpu.CompilerParams(needs_layout_passes=False, use_tc_tiling_on_sc=False),
        scratch_types=[
            pltpu.VMEM((NLOC,), jnp.int32),
            pltpu.VMEM((NLOC, 64), jnp.float32),
            pltpu.VMEM((64, 256), jnp.float32),
            pltpu.SemaphoreType.DMA,
        ],
    )


_bucket = _make_bucket()
_prop128 = _make_prop(128)
_prop64 = _make_prop(64)
_pool = _make_pool()

_BR = 256
_GRID = NPAD // _BR


def _blk(w):
    return pl.BlockSpec((_BR, w), lambda i: (i, 0))


def _full(shape):
    nd = len(shape)
    return pl.BlockSpec(shape, lambda i, _n=nd: (0,) * _n)


def _t0_body(deg_ref, x_ref, dis_ref, dis64_ref, u_ref):
    deg = deg_ref[:, 0:1]
    dis = jnp.where(deg > 0, lax.rsqrt(jnp.maximum(deg, 1e-12)), 0.0)
    dis64_ref[...] = jnp.broadcast_to(dis, (_BR, 64))
    dis = jnp.broadcast_to(dis, (_BR, 128))
    dis_ref[...] = dis
    u_ref[...] = dis * x_ref[...]


def _t0(deg16, xp):
    return pl.pallas_call(
        _t0_body,
        grid=(_GRID,),
        in_specs=[_blk(16), _blk(128)],
        out_specs=[_blk(128), _blk(64), _blk(128)],
        out_shape=[jax.ShapeDtypeStruct((NPAD, 128), jnp.float32),
                   jax.ShapeDtypeStruct((NPAD, 64), jnp.float32),
                   jax.ShapeDtypeStruct((NPAD, 128), jnp.float32)],
    )(deg16, xp)


def _t1_body(s1_ref, dis_ref, u_ref):
    dis = dis_ref[...]
    u_ref[...] = -(dis * dis) * s1_ref[...]


def _t1(S1, dis, D):
    return pl.pallas_call(
        _t1_body,
        grid=(_GRID,),
        in_specs=[_blk(D), _blk(D)],
        out_specs=_blk(D),
        out_shape=jax.ShapeDtypeStruct((NPAD, D), jnp.float32),
    )(S1, dis)


def _layer_math(h, S1, S2, dis, W, b):
    Tx1 = -dis * S1
    Tx2 = -2.0 * dis * S2 - h
    o = (jnp.dot(h, W[0], preferred_element_type=jnp.float32)
         + jnp.dot(Tx1, W[1], preferred_element_type=jnp.float32)
         + jnp.dot(Tx2, W[2], preferred_element_type=jnp.float32) + b)
    return jnp.where(o >= 0, o, ALPHA * o)


def _t2_body(emit_u, h_ref, s1_ref, s2_ref, dis_ref, w_ref, b_ref,
             out_ref, *rest):
    out = _layer_math(h_ref[...], s1_ref[...], s2_ref[...], dis_ref[...],
                      w_ref[...], b_ref[...])
    out_ref[...] = out
    if emit_u:
        rest[0][...] = dis_ref[:, 0:64] * out


def _t2(h, S1, S2, dis, W, b, D, emit_u):
    n_out = 2 if emit_u else 1
    outs = pl.pallas_call(
        functools.partial(_t2_body, emit_u),
        grid=(_GRID,),
        in_specs=[_blk(D), _blk(D), _blk(D), _blk(D),
                  _full((3, D, 64)), _full((1, 64))],
        out_specs=[_blk(64)] * n_out,
        out_shape=[jax.ShapeDtypeStruct((NPAD, 64), jnp.float32)] * n_out,
    )(h, S1, S2, dis, W, b)
    return outs if emit_u else outs[0]


def _t2_skip_body(h_ref, s1_ref, s2_ref, dis_ref, w_ref, b_ref, hs_ref,
                  x2_ref, x3_ref, u_ref):
    x2 = _layer_math(h_ref[...], s1_ref[...], s2_ref[...], dis_ref[...],
                     w_ref[...], b_ref[...])
    x3 = hs_ref[...] + x2
    x2_ref[...] = x2
    x3_ref[...] = x3
    u_ref[...] = dis_ref[...] * x3


def _t2_skip(h, S1, S2, dis, W, b, hskip):
    return pl.pallas_call(
        _t2_skip_body,
        grid=(_GRID,),
        in_specs=[_blk(64)] * 4 + [_full((3, 64, 64)), _full((1, 64)),
                                   _blk(64)],
        out_specs=[_blk(64)] * 3,
        out_shape=[jax.ShapeDtypeStruct((NPAD, 64), jnp.float32)] * 3,
    )(h, S1, S2, dis, W, b, hskip)


def _finish_body(part_ref, w1_ref, b1_ref, g_ref, be_ref, w2_ref, b2_ref,
                 out_ref):
    pooled = jnp.max(part_ref[...], axis=0)
    h = jnp.dot(pooled, w1_ref[...], preferred_element_type=jnp.float32)
    h = h + b1_ref[...]
    h = g_ref[...] * h * (1.0 / jnp.sqrt(1.0 + EPS)) + be_ref[...]
    h = jnp.maximum(h, 0.0)
    out_ref[...] = (jnp.dot(h, w2_ref[...],
                            preferred_element_type=jnp.float32) + b2_ref[...])


def _finish(part, W1, b1, gamma, beta, W2, b2):
    return pl.pallas_call(
        _finish_body,
        grid=(1,),
        in_specs=[_full((NW, 64, 256)), _full((256, 64)), _full((1, 64)),
                  _full((1, 64)), _full((1, 64)), _full((64, 10)),
                  _full((1, 10))],
        out_specs=_full((64, 10)),
        out_shape=jax.ShapeDtypeStruct((64, 10), jnp.float32),
    )(part, W1, b1, gamma, beta, W2, b2)


def kernel(x, edge_index, batch, W_init, b_init, W_head, b_head,
           W_body, b_body, W_tail, b_tail, W1, b1, gamma, beta, W2, b2):
    src = edge_index[0]
    dst = edge_index[1]
    xp = jnp.pad(x, ((0, NPAD - N), (0, 0)))
    batchp = jnp.pad(batch, (0, NPAD - N))

    packed, counts, deg_flat = _bucket(src, dst)
    deg16 = deg_flat.reshape(NPAD, 16)
    dis, dis64, u0 = _t0(deg16, xp)

    def cheb(h, u_in, W, b, D, prop, emit_u):
        d = dis if D == 128 else dis64
        S1 = prop(u_in, packed, counts)
        u1 = _t1(S1, d, D)
        S2 = prop(u1, packed, counts)
        return _t2(h, S1, S2, d, W, b.reshape(1, 64), D, emit_u), S1, S2

    (h0, u_h0), _, _ = cheb(xp, u0, W_init, b_init, 128, _prop128, True)
    (x1, u_x1), _, _ = cheb(h0, u_h0, W_head, b_head, 64, _prop64, True)
    S1 = _prop64(u_x1, packed, counts)
    u1 = _t1(S1, dis64, 64)
    S2 = _prop64(u1, packed, counts)
    x2, x3, u_x3 = _t2_skip(x1, S1, S2, dis64, W_body,
                            b_body.reshape(1, 64), h0)
    x4, _, _ = cheb(x3, u_x3, W_tail, b_tail, 64, _prop64, False)

    part = _pool(x4, x1, x2, x3, batchp)
    return _finish(part, W1, b1.reshape(1, 64), gamma.reshape(1, 64),
                   beta.reshape(1, 64), W2, b2.reshape(1, 10))


# speedup vs baseline: 4.0136x; 1.0897x over previous
import functools

import jax
import jax.numpy as jnp
from jax import lax
from jax.experimental import pallas as pl
from jax.experimental.pallas import tpu as pltpu
from jax.experimental.pallas import tpu_sc as plsc

N = 10000
E = 320000
NC = 2
NS = 16
NW = NC * NS
NLOC = 320
NPAD = NW * NLOC
EPAD = E + 4096
FLUSH = 2048
BUF = FLUSH + 16
F_CH = 6400
F_NCH = E // F_CH
DUMMY_ROW = NLOC
ACC_ROWS = NLOC + 16
DUMMY_PK = DUMMY_ROW << 14
ALPHA = 0.01
EPS = 1e-5
NEG_INF = float("-inf")

_MESH = dict(core_axis_name="c", subcore_axis_name="s", num_cores=NC,
             num_subcores=NS)


def _wid():
    return lax.axis_index("s") * NC + lax.axis_index("c")


def _mo8(v):
    return pl.multiple_of(v, 8)


def _bucket_body(src_hbm, dst_hbm, packed_hbm, counts_hbm, deg_hbm,
                 src_v0, dst_v0, src_v1, dst_v1, buf, cnt16, degacc, pkv,
                 semA, semB):
    wid = _wid()
    lo = wid * NLOC
    pbase = wid * EPAD

    def scan(src_v, dst_v, carry):
        def inner(j, carry):
            cnt, goff = carry
            sv = src_v[pl.ds(j * 16, 16)]
            dv = dst_v[pl.ds(j * 16, 16)]
            m = (dv >= lo) & (dv < lo + NLOC)
            pk = ((dv - lo) << 14) | sv
            incl = plsc.cumsum(m.astype(jnp.int32))
            plsc.store_scatter(buf, [cnt + incl - 1], pk, mask=m)
            cnt = cnt + incl[15]

            def flush(args):
                c, g = args
                pltpu.sync_copy(buf.at[pl.ds(0, FLUSH)],
                                packed_hbm.at[pl.ds(_mo8(pbase + g), FLUSH)])
                tail = buf[pl.ds(FLUSH, 16)]
                buf[pl.ds(0, 16)] = tail
                return c - FLUSH, g + FLUSH

            return lax.cond(cnt >= FLUSH, flush, lambda a: a, (cnt, goff))

        return lax.fori_loop(0, F_CH // 16, inner, carry)

    def start_load(c, src_v, dst_v, sem):
        off = _mo8(c * F_CH)
        pltpu.async_copy(src_hbm.at[pl.ds(off, F_CH)], src_v, sem)
        pltpu.async_copy(dst_hbm.at[pl.ds(off, F_CH)], dst_v, sem)

    def wait_load(src_v, dst_v, sem):
        pltpu.make_async_copy(src_hbm.at[pl.ds(0, F_CH)], src_v, sem).wait()
        pltpu.make_async_copy(dst_hbm.at[pl.ds(0, F_CH)], dst_v, sem).wait()

    start_load(0, src_v0, dst_v0, semA)

    def pair(t, carry):
        start_load(2 * t + 1, src_v1, dst_v1, semB)
        wait_load(src_v0, dst_v0, semA)
        carry = scan(src_v0, dst_v0, carry)
        start_load(jnp.minimum(2 * t + 2, F_NCH - 1), src_v0, dst_v0, semA)
        wait_load(src_v1, dst_v1, semB)
        return scan(src_v1, dst_v1, carry)

    cnt, goff = lax.fori_loop(0, F_NCH // 2, pair,
                              (jnp.int32(0), jnp.int32(0)))
    wait_load(src_v0, dst_v0, semA)
    pltpu.sync_copy(buf.at[pl.ds(0, FLUSH)],
                    packed_hbm.at[pl.ds(_mo8(pbase + goff), FLUSH)])
    cnt = goff + cnt
    cnt16[...] = jnp.full((16,), cnt, jnp.int32)
    pltpu.sync_copy(cnt16, counts_hbm.at[wid])

    zeros = jnp.zeros((16,), jnp.float32)

    def zbody(i, _):
        degacc[pl.ds(i * 16, 16)] = zeros
        return 0

    lax.fori_loop(0, ACC_ROWS, zbody, 0)
    ones = jnp.ones((16,), jnp.float32)

    def degbody(c, _):
        pltpu.sync_copy(packed_hbm.at[pl.ds(_mo8(pbase + c * 1024), 1024)],
                        pkv)

        def degvec(j, _):
            pos = c * 1024 + j * 16 + lax.iota(jnp.int32, 16)
            v = pkv[pl.ds(j * 16, 16)]
            v = jnp.where(pos < cnt, v, jnp.int32(DUMMY_PK))
            dloc = v >> 14
            for l in range(16):
                dl = dloc[l]
                plsc.addupdate(degacc.at[pl.ds(dl * 16, 16)], ones)
            return 0

        lax.fori_loop(0, 64, degvec, 0)
        return 0

    lax.fori_loop(0, (cnt + 1023) // 1024, degbody, 0)
    pltpu.sync_copy(degacc.at[pl.ds(0, NLOC * 16)],
                    deg_hbm.at[pl.ds(_mo8(wid * NLOC * 16), NLOC * 16)])


def _make_bucket():
    return pl.kernel(
        _bucket_body,
        out_type=(
            jax.ShapeDtypeStruct((NW * EPAD,), jnp.int32),
            jax.ShapeDtypeStruct((NW, 16), jnp.int32),
            jax.ShapeDtypeStruct((NPAD * 16,), jnp.float32),
        ),
        mesh=plsc.VectorSubcoreMesh(**_MESH),
        compiler_params=pltpu.CompilerParams(needs_layout_passes=False, use_tc_tiling_on_sc=False),
        scratch_types=[
            pltpu.VMEM((F_CH,), jnp.int32),
            pltpu.VMEM((F_CH,), jnp.int32),
            pltpu.VMEM((F_CH,), jnp.int32),
            pltpu.VMEM((F_CH,), jnp.int32),
            pltpu.VMEM((BUF,), jnp.int32),
            pltpu.VMEM((16,), jnp.int32),
            pltpu.VMEM((ACC_ROWS * 16,), jnp.float32),
            pltpu.VMEM((1024,), jnp.int32),
            pltpu.SemaphoreType.DMA,
            pltpu.SemaphoreType.DMA,
        ],
    )


def _prop_body(D, CH, y_hbm, packed_hbm, counts_hbm, out_hbm,
               pkv0, pkv1, sidx0, sidx1, didx0, didx1, rows0, rows1, acc,
               cntv, sem0, sem1, psem0, psem1):
    wid = _wid()
    lo = wid * NLOC
    pbase = wid * EPAD
    pltpu.sync_copy(counts_hbm.at[wid], cntv)
    cnt = cntv[pl.ds(0, 16)][0]
    zeros = jnp.zeros((16,), jnp.float32)
    ng = D // 16
    nv = CH // 16

    def zbody(r, _):
        for g in range(ng):
            acc[r, pl.ds(g * 16, 16)] = zeros
        return 0

    lax.fori_loop(0, ACC_ROWS, zbody, 0)

    def start_pk(c, pkv, psem):
        pltpu.async_copy(packed_hbm.at[pl.ds(_mo8(pbase + c * CH), CH)],
                         pkv, psem)

    def wait_pk(pkv, psem):
        pltpu.make_async_copy(packed_hbm.at[pl.ds(0, CH)], pkv, psem).wait()

    def unpack(c, pkv, sidx, didx):
        cbase = c * CH
        for j in range(nv):
            pos = cbase + j * 16 + lax.iota(jnp.int32, 16)
            v = pkv[pl.ds(j * 16, 16)]
            v = jnp.where(pos < cnt, v, jnp.int32(DUMMY_PK))
            didx[pl.ds(j * 16, 16)] = v >> 14
            sidx[pl.ds(j * 16, 16)] = v & 0x3FFF

    def accumulate(didx, rows):
        for j in range(nv):
            dloc = didx[pl.ds(j * 16, 16)]
            for l in range(16):
                e = j * 16 + l
                dl = dloc[l]
                for g in range(ng):
                    plsc.addupdate(acc.at[dl, pl.ds(g * 16, 16)],
                                   rows[e, pl.ds(g * 16, 16)])

    nch = (cnt + CH - 1) // CH
    nch2 = (nch + 1) // 2
    start_pk(0, pkv0, psem0)
    start_pk(1, pkv1, psem1)
    wait_pk(pkv0, psem0)
    unpack(0, pkv0, sidx0, didx0)
    pltpu.async_copy(y_hbm.at[sidx0], rows0, sem0)
    start_pk(2, pkv0, psem0)

    def pair(t, _):
        c = 2 * t
        wait_pk(pkv1, psem1)
        unpack(c + 1, pkv1, sidx1, didx1)
        pltpu.async_copy(y_hbm.at[sidx1], rows1, sem1)
        start_pk(c + 3, pkv1, psem1)
        pltpu.make_async_copy(y_hbm.at[sidx0], rows0, sem0).wait()
        accumulate(didx0, rows0)
        wait_pk(pkv0, psem0)
        unpack(c + 2, pkv0, sidx0, didx0)
        pltpu.async_copy(y_hbm.at[sidx0], rows0, sem0)
        start_pk(c + 4, pkv0, psem0)
        pltpu.make_async_copy(y_hbm.at[sidx1], rows1, sem1).wait()
        accumulate(didx1, rows1)
        return 0

    lax.fori_loop(0, nch2, pair, 0)
    pltpu.make_async_copy(y_hbm.at[sidx0], rows0, sem0).wait()
    wait_pk(pkv0, psem0)
    wait_pk(pkv1, psem1)
    pltpu.sync_copy(acc.at[pl.ds(0, NLOC)], out_hbm.at[pl.ds(_mo8(lo), NLOC)])


def _make_prop(D):
    CH = 8192 // D
    return pl.kernel(
        functools.partial(_prop_body, D, CH),
        out_type=jax.ShapeDtypeStruct((NPAD, D), jnp.float32),
        mesh=plsc.VectorSubcoreMesh(**_MESH),
        compiler_params=pltpu.CompilerParams(needs_layout_passes=False, use_tc_tiling_on_sc=False),
        scratch_types=[
            pltpu.VMEM((CH,), jnp.int32),
            pltpu.VMEM((CH,), jnp.int32),
            pltpu.VMEM((CH,), jnp.int32),
            pltpu.VMEM((CH,), jnp.int32),
            pltpu.VMEM((CH,), jnp.int32),
            pltpu.VMEM((CH,), jnp.int32),
            pltpu.VMEM((CH, D), jnp.float32),
            pltpu.VMEM((CH, D), jnp.float32),
            pltpu.VMEM((ACC_ROWS, D), jnp.float32),
            pltpu.VMEM((16,), jnp.int32),
            pltpu.SemaphoreType.DMA,
            pltpu.SemaphoreType.DMA,
            pltpu.SemaphoreType.DMA,
            pltpu.SemaphoreType.DMA,
        ],
    )


def _pool_body(x4, x1, x2, x3, batch_hbm, out_hbm, bv, rowbuf, acc, sem):
    wid = _wid()
    lo = wid * NLOC
    nrows = jnp.minimum(jnp.int32(NLOC), jnp.int32(N) - lo)
    ninf = jnp.full((16,), NEG_INF, jnp.float32)

    def ibody(r, _):
        for g in range(16):
            acc[r, pl.ds(g * 16, 16)] = ninf
        return 0

    lax.fori_loop(0, 64, ibody, 0)
    pltpu.sync_copy(batch_hbm.at[pl.ds(_mo8(lo), NLOC)], bv)
    for ai, arr in enumerate((x4, x1, x2, x3)):
        pltpu.sync_copy(arr.at[pl.ds(_mo8(lo), NLOC)], rowbuf)

        def rbody(j, _):
            bvec = bv[pl.ds(j * 16, 16)]
            for l in range(16):
                g = bvec[l]
                r = j * 16 + l
                for fg in range(4):
                    col = ai * 64 + fg * 16
                    cur = acc[g, pl.ds(col, 16)]
                    acc[g, pl.ds(col, 16)] = jnp.maximum(
                        cur, rowbuf[r, pl.ds(fg * 16, 16)])
            return 0

        lax.fori_loop(0, nrows // 16, rbody, 0)
    pltpu.sync_copy(acc, out_hbm.at[wid])


def _make_pool():
    return pl.kernel(
        _pool_body,
        out_type=jax.ShapeDtypeStruct((NW, 64, 256), jnp.float32),
        mesh=plsc.VectorSubcoreMesh(**_MESH),
        compiler_params=pltpu.CompilerParams(needs_layout_passes=False, use_tc_tiling_on_sc=False),
        scratch_types=[
            pltpu.VMEM((NLOC,), jnp.int32),
            pltpu.VMEM((NLOC, 64), jnp.float32),
            pltpu.VMEM((64, 256), jnp.float32),
            pltpu.SemaphoreType.DMA,
        ],
    )


_bucket = _make_bucket()
_prop128 = _make_prop(128)
_prop64 = _make_prop(64)
_pool = _make_pool()

_BR = 256
_GRID = NPAD // _BR


def _blk(w):
    return pl.BlockSpec((_BR, w), lambda i: (i, 0))


def _full(shape):
    nd = len(shape)
    return pl.BlockSpec(shape, lambda i, _n=nd: (0,) * _n)


def _t0_body(deg_ref, x_ref, dis_ref, dis64_ref, u_ref):
    deg = deg_ref[:, 0:1]
    dis = jnp.where(deg > 0, lax.rsqrt(jnp.maximum(deg, 1e-12)), 0.0)
    dis64_ref[...] = jnp.broadcast_to(dis, (_BR, 64))
    dis = jnp.broadcast_to(dis, (_BR, 128))
    dis_ref[...] = dis
    u_ref[...] = dis * x_ref[...]


def _t0(deg16, xp):
    return pl.pallas_call(
        _t0_body,
        grid=(_GRID,),
        in_specs=[_blk(16), _blk(128)],
        out_specs=[_blk(128), _blk(64), _blk(128)],
        out_shape=[jax.ShapeDtypeStruct((NPAD, 128), jnp.float32),
                   jax.ShapeDtypeStruct((NPAD, 64), jnp.float32),
                   jax.ShapeDtypeStruct((NPAD, 128), jnp.float32)],
    )(deg16, xp)


def _t1_body(s1_ref, dis_ref, u_ref):
    dis = dis_ref[...]
    u_ref[...] = -(dis * dis) * s1_ref[...]


def _t1(S1, dis, D):
    return pl.pallas_call(
        _t1_body,
        grid=(_GRID,),
        in_specs=[_blk(D), _blk(D)],
        out_specs=_blk(D),
        out_shape=jax.ShapeDtypeStruct((NPAD, D), jnp.float32),
    )(S1, dis)


def _layer_math(h, S1, S2, dis, W, b):
    Tx1 = -dis * S1
    Tx2 = -2.0 * dis * S2 - h
    o = (jnp.dot(h, W[0], preferred_element_type=jnp.float32)
         + jnp.dot(Tx1, W[1], preferred_element_type=jnp.float32)
         + jnp.dot(Tx2, W[2], preferred_element_type=jnp.float32) + b)
    return jnp.where(o >= 0, o, ALPHA * o)


def _t2_body(emit_u, h_ref, s1_ref, s2_ref, dis_ref, w_ref, b_ref,
             out_ref, *rest):
    out = _layer_math(h_ref[...], s1_ref[...], s2_ref[...], dis_ref[...],
                      w_ref[...], b_ref[...])
    out_ref[...] = out
    if emit_u:
        rest[0][...] = dis_ref[:, 0:64] * out


def _t2(h, S1, S2, dis, W, b, D, emit_u):
    n_out = 2 if emit_u else 1
    outs = pl.pallas_call(
        functools.partial(_t2_body, emit_u),
        grid=(_GRID,),
        in_specs=[_blk(D), _blk(D), _blk(D), _blk(D),
                  _full((3, D, 64)), _full((1, 64))],
        out_specs=[_blk(64)] * n_out,
        out_shape=[jax.ShapeDtypeStruct((NPAD, 64), jnp.float32)] * n_out,
    )(h, S1, S2, dis, W, b)
    return outs if emit_u else outs[0]


def _t2_skip_body(h_ref, s1_ref, s2_ref, dis_ref, w_ref, b_ref, hs_ref,
                  x2_ref, x3_ref, u_ref):
    x2 = _layer_math(h_ref[...], s1_ref[...], s2_ref[...], dis_ref[...],
                     w_ref[...], b_ref[...])
    x3 = hs_ref[...] + x2
    x2_ref[...] = x2
    x3_ref[...] = x3
    u_ref[...] = dis_ref[...] * x3


def _t2_skip(h, S1, S2, dis, W, b, hskip):
    return pl.pallas_call(
        _t2_skip_body,
        grid=(_GRID,),
        in_specs=[_blk(64)] * 4 + [_full((3, 64, 64)), _full((1, 64)),
                                   _blk(64)],
        out_specs=[_blk(64)] * 3,
        out_shape=[jax.ShapeDtypeStruct((NPAD, 64), jnp.float32)] * 3,
    )(h, S1, S2, dis, W, b, hskip)


def _finish_body(part_ref, w1_ref, b1_ref, g_ref, be_ref, w2_ref, b2_ref,
                 out_ref):
    pooled = jnp.max(part_ref[...], axis=0)
    h = jnp.dot(pooled, w1_ref[...], preferred_element_type=jnp.float32)
    h = h + b1_ref[...]
    h = g_ref[...] * h * (1.0 / jnp.sqrt(1.0 + EPS)) + be_ref[...]
    h = jnp.maximum(h, 0.0)
    out_ref[...] = (jnp.dot(h, w2_ref[...],
                            preferred_element_type=jnp.float32) + b2_ref[...])


def _finish(part, W1, b1, gamma, beta, W2, b2):
    return pl.pallas_call(
        _finish_body,
        grid=(1,),
        in_specs=[_full((NW, 64, 256)), _full((256, 64)), _full((1, 64)),
                  _full((1, 64)), _full((1, 64)), _full((64, 10)),
                  _full((1, 10))],
        out_specs=_full((64, 10)),
        out_shape=jax.ShapeDtypeStruct((64, 10), jnp.float32),
    )(part, W1, b1, gamma, beta, W2, b2)


def kernel(x, edge_index, batch, W_init, b_init, W_head, b_head,
           W_body, b_body, W_tail, b_tail, W1, b1, gamma, beta, W2, b2):
    src = edge_index[0]
    dst = edge_index[1]
    xp = jnp.pad(x, ((0, NPAD - N), (0, 0)))
    batchp = jnp.pad(batch, (0, NPAD - N))

    packed, counts, deg_flat = _bucket(src, dst)
    deg16 = deg_flat.reshape(NPAD, 16)
    dis, dis64, u0 = _t0(deg16, xp)

    def cheb(h, u_in, W, b, D, prop, emit_u):
        d = dis if D == 128 else dis64
        S1 = prop(u_in, packed, counts)
        u1 = _t1(S1, d, D)
        S2 = prop(u1, packed, counts)
        return _t2(h, S1, S2, d, W, b.reshape(1, 64), D, emit_u), S1, S2

    (h0, u_h0), _, _ = cheb(xp, u0, W_init, b_init, 128, _prop128, True)
    (x1, u_x1), _, _ = cheb(h0, u_h0, W_head, b_head, 64, _prop64, True)
    S1 = _prop64(u_x1, packed, counts)
    u1 = _t1(S1, dis64, 64)
    S2 = _prop64(u1, packed, counts)
    x2, x3, u_x3 = _t2_skip(x1, S1, S2, dis64, W_body,
                            b_body.reshape(1, 64), h0)
    x4, _, _ = cheb(x3, u_x3, W_tail, b_tail, 64, _prop64, False)

    part = _pool(x4, x1, x2, x3, batchp)
    return _finish(part, W1, b1.reshape(1, 64), gamma.reshape(1, 64),
                   beta.reshape(1, 64), W2, b2.reshape(1, 10))


# speedup vs baseline: 4.8401x; 1.2059x over previous
import functools

import jax
import jax.numpy as jnp
from jax import lax
from jax.experimental import pallas as pl
from jax.experimental.pallas import tpu as pltpu
from jax.experimental.pallas import tpu_sc as plsc

N = 10000
E = 320000
NC = 2
NS = 16
NW = NC * NS
NLOC = 320
NPAD = NW * NLOC
EPAD = E + 4096
FLUSH = 2048
BUF = FLUSH + 16
F_CH = 6400
F_NCH = E // F_CH
DUMMY_ROW = NLOC
ACC_ROWS = NLOC + 16
DUMMY_PK = DUMMY_ROW << 14
ALPHA = 0.01
EPS = 1e-5
NEG_INF = float("-inf")

_MESH = dict(core_axis_name="c", subcore_axis_name="s", num_cores=NC,
             num_subcores=NS)


def _wid():
    return lax.axis_index("s") * NC + lax.axis_index("c")


def _mo8(v):
    return pl.multiple_of(v, 8)


def _bucket_body(src_hbm, dst_hbm, packed_hbm, counts_hbm, deg_hbm,
                 src_v0, dst_v0, src_v1, dst_v1, buf, cnt16, degacc, pkv,
                 semA, semB):
    wid = _wid()
    lo = wid * NLOC
    pbase = wid * EPAD

    def scan(src_v, dst_v, carry):
        def inner(j, carry):
            cnt, goff = carry
            sv = src_v[pl.ds(j * 16, 16)]
            dv = dst_v[pl.ds(j * 16, 16)]
            m = (dv >= lo) & (dv < lo + NLOC)
            pk = ((dv - lo) << 14) | sv
            incl = plsc.cumsum(m.astype(jnp.int32))
            plsc.store_scatter(buf, [cnt + incl - 1], pk, mask=m)
            cnt = cnt + incl[15]

            def flush(args):
                c, g = args
                pltpu.sync_copy(buf.at[pl.ds(0, FLUSH)],
                                packed_hbm.at[pl.ds(_mo8(pbase + g), FLUSH)])
                tail = buf[pl.ds(FLUSH, 16)]
                buf[pl.ds(0, 16)] = tail
                return c - FLUSH, g + FLUSH

            return lax.cond(cnt >= FLUSH, flush, lambda a: a, (cnt, goff))

        return lax.fori_loop(0, F_CH // 16, inner, carry)

    def start_load(c, src_v, dst_v, sem):
        off = _mo8(c * F_CH)
        pltpu.async_copy(src_hbm.at[pl.ds(off, F_CH)], src_v, sem)
        pltpu.async_copy(dst_hbm.at[pl.ds(off, F_CH)], dst_v, sem)

    def wait_load(src_v, dst_v, sem):
        pltpu.make_async_copy(src_hbm.at[pl.ds(0, F_CH)], src_v, sem).wait()
        pltpu.make_async_copy(dst_hbm.at[pl.ds(0, F_CH)], dst_v, sem).wait()

    start_load(0, src_v0, dst_v0, semA)

    def pair(t, carry):
        start_load(2 * t + 1, src_v1, dst_v1, semB)
        wait_load(src_v0, dst_v0, semA)
        carry = scan(src_v0, dst_v0, carry)
        start_load(jnp.minimum(2 * t + 2, F_NCH - 1), src_v0, dst_v0, semA)
        wait_load(src_v1, dst_v1, semB)
        return scan(src_v1, dst_v1, carry)

    cnt, goff = lax.fori_loop(0, F_NCH // 2, pair,
                              (jnp.int32(0), jnp.int32(0)))
    wait_load(src_v0, dst_v0, semA)
    pltpu.sync_copy(buf.at[pl.ds(0, FLUSH)],
                    packed_hbm.at[pl.ds(_mo8(pbase + goff), FLUSH)])
    cnt = goff + cnt
    cnt16[...] = jnp.full((16,), cnt, jnp.int32)
    pltpu.sync_copy(cnt16, counts_hbm.at[wid])

    zeros = jnp.zeros((16,), jnp.float32)

    def zbody(i, _):
        degacc[pl.ds(i * 16, 16)] = zeros
        return 0

    lax.fori_loop(0, ACC_ROWS, zbody, 0)
    ones = jnp.ones((16,), jnp.float32)

    def degbody(c, _):
        pltpu.sync_copy(packed_hbm.at[pl.ds(_mo8(pbase + c * 1024), 1024)],
                        pkv)

        def degvec(j, _):
            pos = c * 1024 + j * 16 + lax.iota(jnp.int32, 16)
            v = pkv[pl.ds(j * 16, 16)]
            v = jnp.where(pos < cnt, v, jnp.int32(DUMMY_PK))
            dloc = v >> 14
            for l in range(16):
                dl = dloc[l]
                plsc.addupdate(degacc.at[pl.ds(dl * 16, 16)], ones)
            return 0

        lax.fori_loop(0, 64, degvec, 0)
        return 0

    lax.fori_loop(0, (cnt + 1023) // 1024, degbody, 0)
    pltpu.sync_copy(degacc.at[pl.ds(0, NLOC * 16)],
                    deg_hbm.at[pl.ds(_mo8(wid * NLOC * 16), NLOC * 16)])


def _make_bucket():
    return pl.kernel(
        _bucket_body,
        out_type=(
            jax.ShapeDtypeStruct((NW * EPAD,), jnp.int32),
            jax.ShapeDtypeStruct((NW, 16), jnp.int32),
            jax.ShapeDtypeStruct((NPAD * 16,), jnp.float32),
        ),
        mesh=plsc.VectorSubcoreMesh(**_MESH),
        compiler_params=pltpu.CompilerParams(needs_layout_passes=False, use_tc_tiling_on_sc=False),
        scratch_types=[
            pltpu.VMEM((F_CH,), jnp.int32),
            pltpu.VMEM((F_CH,), jnp.int32),
            pltpu.VMEM((F_CH,), jnp.int32),
            pltpu.VMEM((F_CH,), jnp.int32),
            pltpu.VMEM((BUF,), jnp.int32),
            pltpu.VMEM((16,), jnp.int32),
            pltpu.VMEM((ACC_ROWS * 16,), jnp.float32),
            pltpu.VMEM((1024,), jnp.int32),
            pltpu.SemaphoreType.DMA,
            pltpu.SemaphoreType.DMA,
        ],
    )


def _prop_body(D, CH, use_spm, y_hbm, packed_hbm, counts_hbm, out_hbm,
               pkv0, pkv1, sidx0, sidx1, didx0, didx1, rows0, rows1, acc,
               cntv, *rest):
    if use_spm:
        yspm, sem0, sem1, psem0, psem1 = rest
    else:
        sem0, sem1, psem0, psem1 = rest
        yspm = None
    wid = _wid()
    lo = wid * NLOC
    pbase = wid * EPAD
    if use_spm:
        sid = lax.axis_index("s")
        seg = NPAD // NS
        pltpu.sync_copy(y_hbm.at[pl.ds(_mo8(sid * seg), seg)],
                        yspm.at[pl.ds(_mo8(sid * seg), seg)])
    ysrc = yspm if use_spm else y_hbm
    pltpu.sync_copy(counts_hbm.at[wid], cntv)
    cnt = cntv[pl.ds(0, 16)][0]
    if use_spm:
        plsc.subcore_barrier()
    zeros = jnp.zeros((16,), jnp.float32)
    ng = D // 16
    nv = CH // 16

    def zbody(r, _):
        for g in range(ng):
            acc[r, pl.ds(g * 16, 16)] = zeros
        return 0

    lax.fori_loop(0, ACC_ROWS, zbody, 0)

    def start_pk(c, pkv, psem):
        pltpu.async_copy(packed_hbm.at[pl.ds(_mo8(pbase + c * CH), CH)],
                         pkv, psem)

    def wait_pk(pkv, psem):
        pltpu.make_async_copy(packed_hbm.at[pl.ds(0, CH)], pkv, psem).wait()

    def unpack(c, pkv, sidx, didx):
        cbase = c * CH
        for j in range(nv):
            pos = cbase + j * 16 + lax.iota(jnp.int32, 16)
            v = pkv[pl.ds(j * 16, 16)]
            v = jnp.where(pos < cnt, v, jnp.int32(DUMMY_PK))
            didx[pl.ds(j * 16, 16)] = v >> 14
            sidx[pl.ds(j * 16, 16)] = v & 0x3FFF

    def accumulate(didx, rows):
        for j in range(nv):
            dloc = didx[pl.ds(j * 16, 16)]
            for l in range(16):
                e = j * 16 + l
                dl = dloc[l]
                for g in range(ng):
                    plsc.addupdate(acc.at[dl, pl.ds(g * 16, 16)],
                                   rows[e, pl.ds(g * 16, 16)])

    nch = (cnt + CH - 1) // CH
    nch2 = (nch + 1) // 2
    start_pk(0, pkv0, psem0)
    start_pk(1, pkv1, psem1)
    wait_pk(pkv0, psem0)
    unpack(0, pkv0, sidx0, didx0)
    pltpu.async_copy(y_hbm.at[sidx0], rows0, sem0)
    start_pk(2, pkv0, psem0)

    def pair(t, _):
        c = 2 * t
        wait_pk(pkv1, psem1)
        unpack(c + 1, pkv1, sidx1, didx1)
        pltpu.async_copy(ysrc.at[sidx1], rows1, sem1)
        start_pk(c + 3, pkv1, psem1)
        pltpu.make_async_copy(y_hbm.at[sidx0], rows0, sem0).wait()
        accumulate(didx0, rows0)
        wait_pk(pkv0, psem0)
        unpack(c + 2, pkv0, sidx0, didx0)
        pltpu.async_copy(ysrc.at[sidx0], rows0, sem0)
        start_pk(c + 4, pkv0, psem0)
        pltpu.make_async_copy(y_hbm.at[sidx1], rows1, sem1).wait()
        accumulate(didx1, rows1)
        return 0

    lax.fori_loop(0, nch2, pair, 0)
    pltpu.make_async_copy(y_hbm.at[sidx0], rows0, sem0).wait()
    wait_pk(pkv0, psem0)
    wait_pk(pkv1, psem1)
    pltpu.sync_copy(acc.at[pl.ds(0, NLOC)], out_hbm.at[pl.ds(_mo8(lo), NLOC)])


def _make_prop(D):
    CH = 8192 // D
    use_spm = D == 64
    return pl.kernel(
        functools.partial(_prop_body, D, CH, use_spm),
        out_type=jax.ShapeDtypeStruct((NPAD, D), jnp.float32),
        mesh=plsc.VectorSubcoreMesh(**_MESH),
        compiler_params=pltpu.CompilerParams(needs_layout_passes=False, use_tc_tiling_on_sc=False),
        scratch_types=[
            pltpu.VMEM((CH,), jnp.int32),
            pltpu.VMEM((CH,), jnp.int32),
            pltpu.VMEM((CH,), jnp.int32),
            pltpu.VMEM((CH,), jnp.int32),
            pltpu.VMEM((CH,), jnp.int32),
            pltpu.VMEM((CH,), jnp.int32),
            pltpu.VMEM((CH, D), jnp.float32),
            pltpu.VMEM((CH, D), jnp.float32),
            pltpu.VMEM((ACC_ROWS, D), jnp.float32),
            pltpu.VMEM((16,), jnp.int32),
        ] + ([pltpu.VMEM_SHARED((NPAD, D), jnp.float32)] if use_spm else [])
        + [
            pltpu.SemaphoreType.DMA,
            pltpu.SemaphoreType.DMA,
            pltpu.SemaphoreType.DMA,
            pltpu.SemaphoreType.DMA,
        ],
    )


def _pool_body(x4, x1, x2, x3, batch_hbm, out_hbm, bv, rowbuf, acc, sem):
    wid = _wid()
    lo = wid * NLOC
    nrows = jnp.minimum(jnp.int32(NLOC), jnp.int32(N) - lo)
    ninf = jnp.full((16,), NEG_INF, jnp.float32)

    def ibody(r, _):
        for g in range(16):
            acc[r, pl.ds(g * 16, 16)] = ninf
        return 0

    lax.fori_loop(0, 64, ibody, 0)
    pltpu.sync_copy(batch_hbm.at[pl.ds(_mo8(lo), NLOC)], bv)
    for ai, arr in enumerate((x4, x1, x2, x3)):
        pltpu.sync_copy(arr.at[pl.ds(_mo8(lo), NLOC)], rowbuf)

        def rbody(j, _):
            bvec = bv[pl.ds(j * 16, 16)]
            for l in range(16):
                g = bvec[l]
                r = j * 16 + l
                for fg in range(4):
                    col = ai * 64 + fg * 16
                    cur = acc[g, pl.ds(col, 16)]
                    acc[g, pl.ds(col, 16)] = jnp.maximum(
                        cur, rowbuf[r, pl.ds(fg * 16, 16)])
            return 0

        lax.fori_loop(0, nrows // 16, rbody, 0)
    pltpu.sync_copy(acc, out_hbm.at[wid])


def _make_pool():
    return pl.kernel(
        _pool_body,
        out_type=jax.ShapeDtypeStruct((NW, 64, 256), jnp.float32),
        mesh=plsc.VectorSubcoreMesh(**_MESH),
        compiler_params=pltpu.CompilerParams(needs_layout_passes=False, use_tc_tiling_on_sc=False),
        scratch_types=[
            pltpu.VMEM((NLOC,), jnp.int32),
            pltpu.VMEM((NLOC, 64), jnp.float32),
            pltpu.VMEM((64, 256), jnp.float32),
            pltpu.SemaphoreType.DMA,
        ],
    )


_bucket = _make_bucket()
_prop128 = _make_prop(128)
_prop64 = _make_prop(64)
_pool = _make_pool()

_BR = 256
_GRID = NPAD // _BR


def _blk(w):
    return pl.BlockSpec((_BR, w), lambda i: (i, 0))


def _full(shape):
    nd = len(shape)
    return pl.BlockSpec(shape, lambda i, _n=nd: (0,) * _n)


def _t0_body(deg_ref, x_ref, dis_ref, dis64_ref, u_ref):
    deg = deg_ref[:, 0:1]
    dis = jnp.where(deg > 0, lax.rsqrt(jnp.maximum(deg, 1e-12)), 0.0)
    dis64_ref[...] = jnp.broadcast_to(dis, (_BR, 64))
    dis = jnp.broadcast_to(dis, (_BR, 128))
    dis_ref[...] = dis
    u_ref[...] = dis * x_ref[...]


def _t0(deg16, xp):
    return pl.pallas_call(
        _t0_body,
        grid=(_GRID,),
        in_specs=[_blk(16), _blk(128)],
        out_specs=[_blk(128), _blk(64), _blk(128)],
        out_shape=[jax.ShapeDtypeStruct((NPAD, 128), jnp.float32),
                   jax.ShapeDtypeStruct((NPAD, 64), jnp.float32),
                   jax.ShapeDtypeStruct((NPAD, 128), jnp.float32)],
    )(deg16, xp)


def _t1_body(s1_ref, dis_ref, u_ref):
    dis = dis_ref[...]
    u_ref[...] = -(dis * dis) * s1_ref[...]


def _t1(S1, dis, D):
    return pl.pallas_call(
        _t1_body,
        grid=(_GRID,),
        in_specs=[_blk(D), _blk(D)],
        out_specs=_blk(D),
        out_shape=jax.ShapeDtypeStruct((NPAD, D), jnp.float32),
    )(S1, dis)


def _layer_math(h, S1, S2, dis, W, b):
    Tx1 = -dis * S1
    Tx2 = -2.0 * dis * S2 - h
    o = (jnp.dot(h, W[0], preferred_element_type=jnp.float32)
         + jnp.dot(Tx1, W[1], preferred_element_type=jnp.float32)
         + jnp.dot(Tx2, W[2], preferred_element_type=jnp.float32) + b)
    return jnp.where(o >= 0, o, ALPHA * o)


def _t2_body(emit_u, h_ref, s1_ref, s2_ref, dis_ref, w_ref, b_ref,
             out_ref, *rest):
    out = _layer_math(h_ref[...], s1_ref[...], s2_ref[...], dis_ref[...],
                      w_ref[...], b_ref[...])
    out_ref[...] = out
    if emit_u:
        rest[0][...] = dis_ref[:, 0:64] * out


def _t2(h, S1, S2, dis, W, b, D, emit_u):
    n_out = 2 if emit_u else 1
    outs = pl.pallas_call(
        functools.partial(_t2_body, emit_u),
        grid=(_GRID,),
        in_specs=[_blk(D), _blk(D), _blk(D), _blk(D),
                  _full((3, D, 64)), _full((1, 64))],
        out_specs=[_blk(64)] * n_out,
        out_shape=[jax.ShapeDtypeStruct((NPAD, 64), jnp.float32)] * n_out,
    )(h, S1, S2, dis, W, b)
    return outs if emit_u else outs[0]


def _t2_skip_body(h_ref, s1_ref, s2_ref, dis_ref, w_ref, b_ref, hs_ref,
                  x2_ref, x3_ref, u_ref):
    x2 = _layer_math(h_ref[...], s1_ref[...], s2_ref[...], dis_ref[...],
                     w_ref[...], b_ref[...])
    x3 = hs_ref[...] + x2
    x2_ref[...] = x2
    x3_ref[...] = x3
    u_ref[...] = dis_ref[...] * x3


def _t2_skip(h, S1, S2, dis, W, b, hskip):
    return pl.pallas_call(
        _t2_skip_body,
        grid=(_GRID,),
        in_specs=[_blk(64)] * 4 + [_full((3, 64, 64)), _full((1, 64)),
                                   _blk(64)],
        out_specs=[_blk(64)] * 3,
        out_shape=[jax.ShapeDtypeStruct((NPAD, 64), jnp.float32)] * 3,
    )(h, S1, S2, dis, W, b, hskip)


def _finish_body(part_ref, w1_ref, b1_ref, g_ref, be_ref, w2_ref, b2_ref,
                 out_ref):
    pooled = jnp.max(part_ref[...], axis=0)
    h = jnp.dot(pooled, w1_ref[...], preferred_element_type=jnp.float32)
    h = h + b1_ref[...]
    h = g_ref[...] * h * (1.0 / jnp.sqrt(1.0 + EPS)) + be_ref[...]
    h = jnp.maximum(h, 0.0)
    out_ref[...] = (jnp.dot(h, w2_ref[...],
                            preferred_element_type=jnp.float32) + b2_ref[...])


def _finish(part, W1, b1, gamma, beta, W2, b2):
    return pl.pallas_call(
        _finish_body,
        grid=(1,),
        in_specs=[_full((NW, 64, 256)), _full((256, 64)), _full((1, 64)),
                  _full((1, 64)), _full((1, 64)), _full((64, 10)),
                  _full((1, 10))],
        out_specs=_full((64, 10)),
        out_shape=jax.ShapeDtypeStruct((64, 10), jnp.float32),
    )(part, W1, b1, gamma, beta, W2, b2)


def kernel(x, edge_index, batch, W_init, b_init, W_head, b_head,
           W_body, b_body, W_tail, b_tail, W1, b1, gamma, beta, W2, b2):
    src = edge_index[0]
    dst = edge_index[1]
    xp = jnp.pad(x, ((0, NPAD - N), (0, 0)))
    batchp = jnp.pad(batch, (0, NPAD - N))

    packed, counts, deg_flat = _bucket(src, dst)
    deg16 = deg_flat.reshape(NPAD, 16)
    dis, dis64, u0 = _t0(deg16, xp)

    def cheb(h, u_in, W, b, D, prop, emit_u):
        d = dis if D == 128 else dis64
        S1 = prop(u_in, packed, counts)
        u1 = _t1(S1, d, D)
        S2 = prop(u1, packed, counts)
        return _t2(h, S1, S2, d, W, b.reshape(1, 64), D, emit_u), S1, S2

    (h0, u_h0), _, _ = cheb(xp, u0, W_init, b_init, 128, _prop128, True)
    (x1, u_x1), _, _ = cheb(h0, u_h0, W_head, b_head, 64, _prop64, True)
    S1 = _prop64(u_x1, packed, counts)
    u1 = _t1(S1, dis64, 64)
    S2 = _prop64(u1, packed, counts)
    x2, x3, u_x3 = _t2_skip(x1, S1, S2, dis64, W_body,
                            b_body.reshape(1, 64), h0)
    x4, _, _ = cheb(x3, u_x3, W_tail, b_tail, 64, _prop64, False)

    part = _pool(x4, x1, x2, x3, batchp)
    return _finish(part, W1, b1.reshape(1, 64), gamma.reshape(1, 64),
                   beta.reshape(1, 64), W2, b2.reshape(1, 10))


# speedup vs baseline: 5.2407x; 1.0828x over previous
import functools

import jax
import jax.numpy as jnp
from jax import lax
from jax.experimental import pallas as pl
from jax.experimental.pallas import tpu as pltpu
from jax.experimental.pallas import tpu_sc as plsc

N = 10000
E = 320000
NC = 2
NS = 16
NW = NC * NS
NLOC = 320
NPAD = NW * NLOC
EPAD = E + 4096
FLUSH = 2048
BUF = FLUSH + 144
F_CH = 6400
F_NCH = E // F_CH
DUMMY_ROW = NLOC
ACC_ROWS = NLOC + 16
DUMMY_PK = DUMMY_ROW << 14
ALPHA = 0.01
EPS = 1e-5
NEG_INF = float("-inf")

_MESH = dict(core_axis_name="c", subcore_axis_name="s", num_cores=NC,
             num_subcores=NS)


def _wid():
    return lax.axis_index("s") * NC + lax.axis_index("c")


def _mo8(v):
    return pl.multiple_of(v, 8)


def _bucket_body(src_hbm, dst_hbm, packed_hbm, counts_hbm, deg_hbm,
                 src_v0, dst_v0, src_v1, dst_v1, buf, cnt16, degacc, pkv,
                 semA, semB):
    wid = _wid()
    lo = wid * NLOC
    pbase = wid * EPAD

    def scan(src_v, dst_v, carry):
        def group(q, carry):
            cnt, goff = carry
            for u in range(8):
                j = q * 8 + u
                sv = src_v[pl.ds(j * 16, 16)]
                dv = dst_v[pl.ds(j * 16, 16)]
                m = (dv >= lo) & (dv < lo + NLOC)
                pk = ((dv - lo) << 14) | sv
                incl = plsc.cumsum(m.astype(jnp.int32))
                plsc.store_scatter(buf, [cnt + incl - 1], pk, mask=m)
                cnt = cnt + incl[15]

            def flush(args):
                c, g = args
                pltpu.sync_copy(buf.at[pl.ds(0, FLUSH)],
                                packed_hbm.at[pl.ds(_mo8(pbase + g), FLUSH)])
                for u in range(9):
                    tail = buf[pl.ds(FLUSH + u * 16, 16)]
                    buf[pl.ds(u * 16, 16)] = tail
                return c - FLUSH, g + FLUSH

            return lax.cond(cnt >= FLUSH, flush, lambda a: a, (cnt, goff))

        return lax.fori_loop(0, F_CH // 128, group, carry)

    def start_load(c, src_v, dst_v, sem):
        off = _mo8(c * F_CH)
        pltpu.async_copy(src_hbm.at[pl.ds(off, F_CH)], src_v, sem)
        pltpu.async_copy(dst_hbm.at[pl.ds(off, F_CH)], dst_v, sem)

    def wait_load(src_v, dst_v, sem):
        pltpu.make_async_copy(src_hbm.at[pl.ds(0, F_CH)], src_v, sem).wait()
        pltpu.make_async_copy(dst_hbm.at[pl.ds(0, F_CH)], dst_v, sem).wait()

    start_load(0, src_v0, dst_v0, semA)

    def pair(t, carry):
        start_load(2 * t + 1, src_v1, dst_v1, semB)
        wait_load(src_v0, dst_v0, semA)
        carry = scan(src_v0, dst_v0, carry)
        start_load(jnp.minimum(2 * t + 2, F_NCH - 1), src_v0, dst_v0, semA)
        wait_load(src_v1, dst_v1, semB)
        return scan(src_v1, dst_v1, carry)

    cnt, goff = lax.fori_loop(0, F_NCH // 2, pair,
                              (jnp.int32(0), jnp.int32(0)))
    wait_load(src_v0, dst_v0, semA)
    pltpu.sync_copy(buf.at[pl.ds(0, FLUSH)],
                    packed_hbm.at[pl.ds(_mo8(pbase + goff), FLUSH)])
    cnt = goff + cnt
    cnt16[...] = jnp.full((16,), cnt, jnp.int32)
    pltpu.sync_copy(cnt16, counts_hbm.at[wid])

    zeros = jnp.zeros((16,), jnp.float32)

    def zbody(i, _):
        degacc[pl.ds(i * 16, 16)] = zeros
        return 0

    lax.fori_loop(0, ACC_ROWS, zbody, 0)
    ones = jnp.ones((16,), jnp.float32)

    def degbody(c, _):
        pltpu.sync_copy(packed_hbm.at[pl.ds(_mo8(pbase + c * 1024), 1024)],
                        pkv)

        def degvec(j, _):
            pos = c * 1024 + j * 16 + lax.iota(jnp.int32, 16)
            v = pkv[pl.ds(j * 16, 16)]
            v = jnp.where(pos < cnt, v, jnp.int32(DUMMY_PK))
            dloc = v >> 14
            for l in range(16):
                dl = dloc[l]
                plsc.addupdate(degacc.at[pl.ds(dl * 16, 16)], ones)
            return 0

        lax.fori_loop(0, 64, degvec, 0)
        return 0

    lax.fori_loop(0, (cnt + 1023) // 1024, degbody, 0)
    pltpu.sync_copy(degacc.at[pl.ds(0, NLOC * 16)],
                    deg_hbm.at[pl.ds(_mo8(wid * NLOC * 16), NLOC * 16)])


def _make_bucket():
    return pl.kernel(
        _bucket_body,
        out_type=(
            jax.ShapeDtypeStruct((NW * EPAD,), jnp.int32),
            jax.ShapeDtypeStruct((NW, 16), jnp.int32),
            jax.ShapeDtypeStruct((NPAD * 16,), jnp.float32),
        ),
        mesh=plsc.VectorSubcoreMesh(**_MESH),
        compiler_params=pltpu.CompilerParams(needs_layout_passes=False, use_tc_tiling_on_sc=False),
        scratch_types=[
            pltpu.VMEM((F_CH,), jnp.int32),
            pltpu.VMEM((F_CH,), jnp.int32),
            pltpu.VMEM((F_CH,), jnp.int32),
            pltpu.VMEM((F_CH,), jnp.int32),
            pltpu.VMEM((BUF,), jnp.int32),
            pltpu.VMEM((16,), jnp.int32),
            pltpu.VMEM((ACC_ROWS * 16,), jnp.float32),
            pltpu.VMEM((1024,), jnp.int32),
            pltpu.SemaphoreType.DMA,
            pltpu.SemaphoreType.DMA,
        ],
    )


def _prop_body(D, CH, use_spm, y_hbm, packed_hbm, counts_hbm, out_hbm,
               pkv0, pkv1, sidx0, sidx1, didx0, didx1, rows0, rows1, acc,
               cntv, *rest):
    if use_spm:
        yspm, sem0, sem1, psem0, psem1 = rest
    else:
        sem0, sem1, psem0, psem1 = rest
        yspm = None
    wid = _wid()
    lo = wid * NLOC
    pbase = wid * EPAD
    if use_spm:
        sid = lax.axis_index("s")
        seg = NPAD // NS
        pltpu.sync_copy(y_hbm.at[pl.ds(_mo8(sid * seg), seg)],
                        yspm.at[pl.ds(_mo8(sid * seg), seg)])
    ysrc = yspm if use_spm else y_hbm
    pltpu.sync_copy(counts_hbm.at[wid], cntv)
    cnt = cntv[pl.ds(0, 16)][0]
    if use_spm:
        plsc.subcore_barrier()
    zeros = jnp.zeros((16,), jnp.float32)
    ng = D // 16
    nv = CH // 16

    def zbody(r, _):
        for g in range(ng):
            acc[r, pl.ds(g * 16, 16)] = zeros
        return 0

    lax.fori_loop(0, ACC_ROWS, zbody, 0)

    def start_pk(c, pkv, psem):
        pltpu.async_copy(packed_hbm.at[pl.ds(_mo8(pbase + c * CH), CH)],
                         pkv, psem)

    def wait_pk(pkv, psem):
        pltpu.make_async_copy(packed_hbm.at[pl.ds(0, CH)], pkv, psem).wait()

    def unpack(c, pkv, sidx, didx):
        cbase = c * CH
        for j in range(nv):
            pos = cbase + j * 16 + lax.iota(jnp.int32, 16)
            v = pkv[pl.ds(j * 16, 16)]
            v = jnp.where(pos < cnt, v, jnp.int32(DUMMY_PK))
            didx[pl.ds(j * 16, 16)] = v >> 14
            sidx[pl.ds(j * 16, 16)] = v & 0x3FFF

    def accumulate(didx, rows):
        for j in range(nv):
            dloc = didx[pl.ds(j * 16, 16)]
            for l in range(16):
                e = j * 16 + l
                dl = dloc[l]
                for g in range(ng):
                    plsc.addupdate(acc.at[dl, pl.ds(g * 16, 16)],
                                   rows[e, pl.ds(g * 16, 16)])

    nch = (cnt + CH - 1) // CH
    nch2 = (nch + 1) // 2
    start_pk(0, pkv0, psem0)
    start_pk(1, pkv1, psem1)
    wait_pk(pkv0, psem0)
    unpack(0, pkv0, sidx0, didx0)
    pltpu.async_copy(y_hbm.at[sidx0], rows0, sem0)
    start_pk(2, pkv0, psem0)

    def pair(t, _):
        c = 2 * t
        wait_pk(pkv1, psem1)
        unpack(c + 1, pkv1, sidx1, didx1)
        pltpu.async_copy(ysrc.at[sidx1], rows1, sem1)
        start_pk(c + 3, pkv1, psem1)
        pltpu.make_async_copy(y_hbm.at[sidx0], rows0, sem0).wait()
        accumulate(didx0, rows0)
        wait_pk(pkv0, psem0)
        unpack(c + 2, pkv0, sidx0, didx0)
        pltpu.async_copy(ysrc.at[sidx0], rows0, sem0)
        start_pk(c + 4, pkv0, psem0)
        pltpu.make_async_copy(y_hbm.at[sidx1], rows1, sem1).wait()
        accumulate(didx1, rows1)
        return 0

    lax.fori_loop(0, nch2, pair, 0)
    pltpu.make_async_copy(y_hbm.at[sidx0], rows0, sem0).wait()
    wait_pk(pkv0, psem0)
    wait_pk(pkv1, psem1)
    pltpu.sync_copy(acc.at[pl.ds(0, NLOC)], out_hbm.at[pl.ds(_mo8(lo), NLOC)])


def _make_prop(D):
    CH = 8192 // D
    use_spm = D == 64
    return pl.kernel(
        functools.partial(_prop_body, D, CH, use_spm),
        out_type=jax.ShapeDtypeStruct((NPAD, D), jnp.float32),
        mesh=plsc.VectorSubcoreMesh(**_MESH),
        compiler_params=pltpu.CompilerParams(needs_layout_passes=False, use_tc_tiling_on_sc=False),
        scratch_types=[
            pltpu.VMEM((CH,), jnp.int32),
            pltpu.VMEM((CH,), jnp.int32),
            pltpu.VMEM((CH,), jnp.int32),
            pltpu.VMEM((CH,), jnp.int32),
            pltpu.VMEM((CH,), jnp.int32),
            pltpu.VMEM((CH,), jnp.int32),
            pltpu.VMEM((CH, D), jnp.float32),
            pltpu.VMEM((CH, D), jnp.float32),
            pltpu.VMEM((ACC_ROWS, D), jnp.float32),
            pltpu.VMEM((16,), jnp.int32),
        ] + ([pltpu.VMEM_SHARED((NPAD, D), jnp.float32)] if use_spm else [])
        + [
            pltpu.SemaphoreType.DMA,
            pltpu.SemaphoreType.DMA,
            pltpu.SemaphoreType.DMA,
            pltpu.SemaphoreType.DMA,
        ],
    )


def _pool_body(x4, x1, x2, x3, batch_hbm, out_hbm, bv, rowbuf, acc, sem):
    wid = _wid()
    lo = wid * NLOC
    nrows = jnp.minimum(jnp.int32(NLOC), jnp.int32(N) - lo)
    ninf = jnp.full((16,), NEG_INF, jnp.float32)

    def ibody(r, _):
        for g in range(16):
            acc[r, pl.ds(g * 16, 16)] = ninf
        return 0

    lax.fori_loop(0, 64, ibody, 0)
    pltpu.sync_copy(batch_hbm.at[pl.ds(_mo8(lo), NLOC)], bv)
    for ai, arr in enumerate((x4, x1, x2, x3)):
        pltpu.sync_copy(arr.at[pl.ds(_mo8(lo), NLOC)], rowbuf)

        def rbody(j, _):
            bvec = bv[pl.ds(j * 16, 16)]
            for l in range(16):
                g = bvec[l]
                r = j * 16 + l
                for fg in range(4):
                    col = ai * 64 + fg * 16
                    cur = acc[g, pl.ds(col, 16)]
                    acc[g, pl.ds(col, 16)] = jnp.maximum(
                        cur, rowbuf[r, pl.ds(fg * 16, 16)])
            return 0

        lax.fori_loop(0, nrows // 16, rbody, 0)
    pltpu.sync_copy(acc, out_hbm.at[wid])


def _make_pool():
    return pl.kernel(
        _pool_body,
        out_type=jax.ShapeDtypeStruct((NW, 64, 256), jnp.float32),
        mesh=plsc.VectorSubcoreMesh(**_MESH),
        compiler_params=pltpu.CompilerParams(needs_layout_passes=False, use_tc_tiling_on_sc=False),
        scratch_types=[
            pltpu.VMEM((NLOC,), jnp.int32),
            pltpu.VMEM((NLOC, 64), jnp.float32),
            pltpu.VMEM((64, 256), jnp.float32),
            pltpu.SemaphoreType.DMA,
        ],
    )


_bucket = _make_bucket()
_prop128 = _make_prop(128)
_prop64 = _make_prop(64)
_pool = _make_pool()

_BR = 256
_GRID = NPAD // _BR


def _blk(w):
    return pl.BlockSpec((_BR, w), lambda i: (i, 0))


def _full(shape):
    nd = len(shape)
    return pl.BlockSpec(shape, lambda i, _n=nd: (0,) * _n)


def _t0_body(deg_ref, x_ref, dis_ref, dis64_ref, u_ref):
    deg = deg_ref[:, 0:1]
    dis = jnp.where(deg > 0, lax.rsqrt(jnp.maximum(deg, 1e-12)), 0.0)
    dis64_ref[...] = jnp.broadcast_to(dis, (_BR, 64))
    dis = jnp.broadcast_to(dis, (_BR, 128))
    dis_ref[...] = dis
    u_ref[...] = dis * x_ref[...]


def _t0(deg16, xp):
    return pl.pallas_call(
        _t0_body,
        grid=(_GRID,),
        in_specs=[_blk(16), _blk(128)],
        out_specs=[_blk(128), _blk(64), _blk(128)],
        out_shape=[jax.ShapeDtypeStruct((NPAD, 128), jnp.float32),
                   jax.ShapeDtypeStruct((NPAD, 64), jnp.float32),
                   jax.ShapeDtypeStruct((NPAD, 128), jnp.float32)],
    )(deg16, xp)


def _t1_body(s1_ref, dis_ref, u_ref):
    dis = dis_ref[...]
    u_ref[...] = -(dis * dis) * s1_ref[...]


def _t1(S1, dis, D):
    return pl.pallas_call(
        _t1_body,
        grid=(_GRID,),
        in_specs=[_blk(D), _blk(D)],
        out_specs=_blk(D),
        out_shape=jax.ShapeDtypeStruct((NPAD, D), jnp.float32),
    )(S1, dis)


def _layer_math(h, S1, S2, dis, W, b):
    Tx1 = -dis * S1
    Tx2 = -2.0 * dis * S2 - h
    o = (jnp.dot(h, W[0], preferred_element_type=jnp.float32)
         + jnp.dot(Tx1, W[1], preferred_element_type=jnp.float32)
         + jnp.dot(Tx2, W[2], preferred_element_type=jnp.float32) + b)
    return jnp.where(o >= 0, o, ALPHA * o)


def _t2_body(emit_u, h_ref, s1_ref, s2_ref, dis_ref, w_ref, b_ref,
             out_ref, *rest):
    out = _layer_math(h_ref[...], s1_ref[...], s2_ref[...], dis_ref[...],
                      w_ref[...], b_ref[...])
    out_ref[...] = out
    if emit_u:
        rest[0][...] = dis_ref[:, 0:64] * out


def _t2(h, S1, S2, dis, W, b, D, emit_u):
    n_out = 2 if emit_u else 1
    outs = pl.pallas_call(
        functools.partial(_t2_body, emit_u),
        grid=(_GRID,),
        in_specs=[_blk(D), _blk(D), _blk(D), _blk(D),
                  _full((3, D, 64)), _full((1, 64))],
        out_specs=[_blk(64)] * n_out,
        out_shape=[jax.ShapeDtypeStruct((NPAD, 64), jnp.float32)] * n_out,
    )(h, S1, S2, dis, W, b)
    return outs if emit_u else outs[0]


def _t2_skip_body(h_ref, s1_ref, s2_ref, dis_ref, w_ref, b_ref, hs_ref,
                  x2_ref, x3_ref, u_ref):
    x2 = _layer_math(h_ref[...], s1_ref[...], s2_ref[...], dis_ref[...],
                     w_ref[...], b_ref[...])
    x3 = hs_ref[...] + x2
    x2_ref[...] = x2
    x3_ref[...] = x3
    u_ref[...] = dis_ref[...] * x3


def _t2_skip(h, S1, S2, dis, W, b, hskip):
    return pl.pallas_call(
        _t2_skip_body,
        grid=(_GRID,),
        in_specs=[_blk(64)] * 4 + [_full((3, 64, 64)), _full((1, 64)),
                                   _blk(64)],
        out_specs=[_blk(64)] * 3,
        out_shape=[jax.ShapeDtypeStruct((NPAD, 64), jnp.float32)] * 3,
    )(h, S1, S2, dis, W, b, hskip)


def _finish_body(part_ref, w1_ref, b1_ref, g_ref, be_ref, w2_ref, b2_ref,
                 out_ref):
    pooled = jnp.max(part_ref[...], axis=0)
    h = jnp.dot(pooled, w1_ref[...], preferred_element_type=jnp.float32)
    h = h + b1_ref[...]
    h = g_ref[...] * h * (1.0 / jnp.sqrt(1.0 + EPS)) + be_ref[...]
    h = jnp.maximum(h, 0.0)
    out_ref[...] = (jnp.dot(h, w2_ref[...],
                            preferred_element_type=jnp.float32) + b2_ref[...])


def _finish(part, W1, b1, gamma, beta, W2, b2):
    return pl.pallas_call(
        _finish_body,
        grid=(1,),
        in_specs=[_full((NW, 64, 256)), _full((256, 64)), _full((1, 64)),
                  _full((1, 64)), _full((1, 64)), _full((64, 10)),
                  _full((1, 10))],
        out_specs=_full((64, 10)),
        out_shape=jax.ShapeDtypeStruct((64, 10), jnp.float32),
    )(part, W1, b1, gamma, beta, W2, b2)


def kernel(x, edge_index, batch, W_init, b_init, W_head, b_head,
           W_body, b_body, W_tail, b_tail, W1, b1, gamma, beta, W2, b2):
    src = edge_index[0]
    dst = edge_index[1]
    xp = jnp.pad(x, ((0, NPAD - N), (0, 0)))
    batchp = jnp.pad(batch, (0, NPAD - N))

    packed, counts, deg_flat = _bucket(src, dst)
    deg16 = deg_flat.reshape(NPAD, 16)
    dis, dis64, u0 = _t0(deg16, xp)

    def cheb(h, u_in, W, b, D, prop, emit_u):
        d = dis if D == 128 else dis64
        S1 = prop(u_in, packed, counts)
        u1 = _t1(S1, d, D)
        S2 = prop(u1, packed, counts)
        return _t2(h, S1, S2, d, W, b.reshape(1, 64), D, emit_u), S1, S2

    (h0, u_h0), _, _ = cheb(xp, u0, W_init, b_init, 128, _prop128, True)
    (x1, u_x1), _, _ = cheb(h0, u_h0, W_head, b_head, 64, _prop64, True)
    S1 = _prop64(u_x1, packed, counts)
    u1 = _t1(S1, dis64, 64)
    S2 = _prop64(u1, packed, counts)
    x2, x3, u_x3 = _t2_skip(x1, S1, S2, dis64, W_body,
                            b_body.reshape(1, 64), h0)
    x4, _, _ = cheb(x3, u_x3, W_tail, b_tail, 64, _prop64, False)

    part = _pool(x4, x1, x2, x3, batchp)
    return _finish(part, W1, b1.reshape(1, 64), gamma.reshape(1, 64),
                   beta.reshape(1, 64), W2, b2.reshape(1, 10))


# speedup vs baseline: 5.8877x; 1.1235x over previous
import functools

import jax
import jax.numpy as jnp
from jax import lax
from jax.experimental import pallas as pl
from jax.experimental.pallas import tpu as pltpu
from jax.experimental.pallas import tpu_sc as plsc

N = 10000
E = 320000
NC = 2
NS = 16
NW = NC * NS
NLOC = 320
NPAD = NW * NLOC
EPAD = E + 4096
FLUSH = 2048
BUF = FLUSH + 144
F_CH = 6400
F_NCH = E // F_CH
DUMMY_ROW = NLOC
ACC_ROWS = NLOC + 16
DUMMY_PK = DUMMY_ROW << 14
ALPHA = 0.01
EPS = 1e-5
NEG_INF = float("-inf")

_MESH = dict(core_axis_name="c", subcore_axis_name="s", num_cores=NC,
             num_subcores=NS)


def _wid():
    return lax.axis_index("s") * NC + lax.axis_index("c")


def _mo8(v):
    return pl.multiple_of(v, 8)


def _bucket_body(src_hbm, dst_hbm, packed_hbm, counts_hbm, deg_hbm,
                 src_v0, dst_v0, src_v1, dst_v1, buf, cnt16, degacc, pkv,
                 semA, semB):
    wid = _wid()
    lo = wid * NLOC
    pbase = wid * EPAD

    def scan(src_v, dst_v, carry):
        def group(q, carry):
            cnt, goff = carry
            for u in range(8):
                j = q * 8 + u
                sv = src_v[pl.ds(j * 16, 16)]
                dv = dst_v[pl.ds(j * 16, 16)]
                m = (dv >= lo) & (dv < lo + NLOC)
                pk = ((dv - lo) << 14) | sv
                incl = plsc.cumsum(m.astype(jnp.int32))
                plsc.store_scatter(buf, [cnt + incl - 1], pk, mask=m)
                cnt = cnt + incl[15]

            def flush(args):
                c, g = args
                pltpu.sync_copy(buf.at[pl.ds(0, FLUSH)],
                                packed_hbm.at[pl.ds(_mo8(pbase + g), FLUSH)])
                for u in range(9):
                    tail = buf[pl.ds(FLUSH + u * 16, 16)]
                    buf[pl.ds(u * 16, 16)] = tail
                return c - FLUSH, g + FLUSH

            return lax.cond(cnt >= FLUSH, flush, lambda a: a, (cnt, goff))

        return lax.fori_loop(0, F_CH // 128, group, carry)

    def start_load(c, src_v, dst_v, sem):
        off = _mo8(c * F_CH)
        pltpu.async_copy(src_hbm.at[pl.ds(off, F_CH)], src_v, sem)
        pltpu.async_copy(dst_hbm.at[pl.ds(off, F_CH)], dst_v, sem)

    def wait_load(src_v, dst_v, sem):
        pltpu.make_async_copy(src_hbm.at[pl.ds(0, F_CH)], src_v, sem).wait()
        pltpu.make_async_copy(dst_hbm.at[pl.ds(0, F_CH)], dst_v, sem).wait()

    start_load(0, src_v0, dst_v0, semA)

    def pair(t, carry):
        start_load(2 * t + 1, src_v1, dst_v1, semB)
        wait_load(src_v0, dst_v0, semA)
        carry = scan(src_v0, dst_v0, carry)
        start_load(jnp.minimum(2 * t + 2, F_NCH - 1), src_v0, dst_v0, semA)
        wait_load(src_v1, dst_v1, semB)
        return scan(src_v1, dst_v1, carry)

    cnt, goff = lax.fori_loop(0, F_NCH // 2, pair,
                              (jnp.int32(0), jnp.int32(0)))
    wait_load(src_v0, dst_v0, semA)
    pltpu.sync_copy(buf.at[pl.ds(0, FLUSH)],
                    packed_hbm.at[pl.ds(_mo8(pbase + goff), FLUSH)])
    cnt = goff + cnt
    cnt16[...] = jnp.full((16,), cnt, jnp.int32)
    pltpu.sync_copy(cnt16, counts_hbm.at[wid])

    zeros = jnp.zeros((16,), jnp.float32)

    def zbody(i, _):
        degacc[pl.ds(i * 16, 16)] = zeros
        return 0

    lax.fori_loop(0, ACC_ROWS, zbody, 0)
    ones = jnp.ones((16,), jnp.float32)

    def degbody(c, _):
        pltpu.sync_copy(packed_hbm.at[pl.ds(_mo8(pbase + c * 1024), 1024)],
                        pkv)

        def degvec(j, _):
            pos = c * 1024 + j * 16 + lax.iota(jnp.int32, 16)
            v = pkv[pl.ds(j * 16, 16)]
            v = jnp.where(pos < cnt, v, jnp.int32(DUMMY_PK))
            dloc = v >> 14
            for l in range(16):
                dl = dloc[l]
                plsc.addupdate(degacc.at[pl.ds(dl * 16, 16)], ones)
            return 0

        lax.fori_loop(0, 64, degvec, 0)
        return 0

    lax.fori_loop(0, (cnt + 1023) // 1024, degbody, 0)
    pltpu.sync_copy(degacc.at[pl.ds(0, NLOC * 16)],
                    deg_hbm.at[pl.ds(_mo8(wid * NLOC * 16), NLOC * 16)])


def _make_bucket():
    return pl.kernel(
        _bucket_body,
        out_type=(
            jax.ShapeDtypeStruct((NW * EPAD,), jnp.int32),
            jax.ShapeDtypeStruct((NW, 16), jnp.int32),
            jax.ShapeDtypeStruct((NPAD * 16,), jnp.float32),
        ),
        mesh=plsc.VectorSubcoreMesh(**_MESH),
        compiler_params=pltpu.CompilerParams(needs_layout_passes=False, use_tc_tiling_on_sc=False),
        scratch_types=[
            pltpu.VMEM((F_CH,), jnp.int32),
            pltpu.VMEM((F_CH,), jnp.int32),
            pltpu.VMEM((F_CH,), jnp.int32),
            pltpu.VMEM((F_CH,), jnp.int32),
            pltpu.VMEM((BUF,), jnp.int32),
            pltpu.VMEM((16,), jnp.int32),
            pltpu.VMEM((ACC_ROWS * 16,), jnp.float32),
            pltpu.VMEM((1024,), jnp.int32),
            pltpu.SemaphoreType.DMA,
            pltpu.SemaphoreType.DMA,
        ],
    )


def _prop_body(D, CH, use_spm, y_hbm, packed_hbm, counts_hbm, out_hbm,
               pkv0, pkv1, sidx0, sidx1, didx0, didx1, rows0, rows1, acc,
               cntv, *rest):
    if use_spm:
        yspm, sem0, sem1, psem0, psem1 = rest
    else:
        sem0, sem1, psem0, psem1 = rest
        yspm = None
    wid = _wid()
    lo = wid * NLOC
    pbase = wid * EPAD
    if use_spm:
        sid = lax.axis_index("s")
        seg = NPAD // NS
        pltpu.sync_copy(y_hbm.at[pl.ds(_mo8(sid * seg), seg)],
                        yspm.at[pl.ds(_mo8(sid * seg), seg)])
    ysrc = yspm if use_spm else y_hbm
    pltpu.sync_copy(counts_hbm.at[wid], cntv)
    cnt = cntv[pl.ds(0, 16)][0]
    if use_spm:
        plsc.subcore_barrier()
    zeros = jnp.zeros((16,), jnp.float32)
    ng = D // 16
    nv = CH // 16

    def zbody(r, _):
        for g in range(ng):
            acc[r, pl.ds(g * 16, 16)] = zeros
        return 0

    lax.fori_loop(0, ACC_ROWS, zbody, 0)

    def start_pk(c, pkv, psem):
        pltpu.async_copy(packed_hbm.at[pl.ds(_mo8(pbase + c * CH), CH)],
                         pkv, psem)

    def wait_pk(pkv, psem):
        pltpu.make_async_copy(packed_hbm.at[pl.ds(0, CH)], pkv, psem).wait()

    def unpack(c, pkv, sidx, didx):
        cbase = c * CH
        for j in range(nv):
            pos = cbase + j * 16 + lax.iota(jnp.int32, 16)
            v = pkv[pl.ds(j * 16, 16)]
            v = jnp.where(pos < cnt, v, jnp.int32(DUMMY_PK))
            didx[pl.ds(j * 16, 16)] = v >> 14
            sidx[pl.ds(j * 16, 16)] = v & 0x3FFF

    def accumulate(didx, rows):
        for j in range(nv):
            dloc = didx[pl.ds(j * 16, 16)]
            for l in range(16):
                e = j * 16 + l
                dl = dloc[l]
                for g in range(ng):
                    plsc.addupdate(acc.at[dl, pl.ds(g * 16, 16)],
                                   rows[e, pl.ds(g * 16, 16)])

    nch = (cnt + CH - 1) // CH
    nch2 = (nch + 1) // 2
    start_pk(0, pkv0, psem0)
    start_pk(1, pkv1, psem1)
    wait_pk(pkv0, psem0)
    unpack(0, pkv0, sidx0, didx0)
    pltpu.async_copy(y_hbm.at[sidx0], rows0, sem0)
    start_pk(2, pkv0, psem0)

    def pair(t, _):
        c = 2 * t
        wait_pk(pkv1, psem1)
        unpack(c + 1, pkv1, sidx1, didx1)
        pltpu.async_copy(ysrc.at[sidx1], rows1, sem1)
        start_pk(c + 3, pkv1, psem1)
        pltpu.make_async_copy(y_hbm.at[sidx0], rows0, sem0).wait()
        accumulate(didx0, rows0)
        wait_pk(pkv0, psem0)
        unpack(c + 2, pkv0, sidx0, didx0)
        pltpu.async_copy(ysrc.at[sidx0], rows0, sem0)
        start_pk(c + 4, pkv0, psem0)
        pltpu.make_async_copy(y_hbm.at[sidx1], rows1, sem1).wait()
        accumulate(didx1, rows1)
        return 0

    lax.fori_loop(0, nch2, pair, 0)
    pltpu.make_async_copy(y_hbm.at[sidx0], rows0, sem0).wait()
    wait_pk(pkv0, psem0)
    wait_pk(pkv1, psem1)
    pltpu.sync_copy(acc.at[pl.ds(0, NLOC)], out_hbm.at[pl.ds(_mo8(lo), NLOC)])


def _make_prop(D):
    CH = 8192 // D
    use_spm = D == 64
    return pl.kernel(
        functools.partial(_prop_body, D, CH, use_spm),
        out_type=jax.ShapeDtypeStruct((NPAD, D), jnp.float32),
        mesh=plsc.VectorSubcoreMesh(**_MESH),
        compiler_params=pltpu.CompilerParams(needs_layout_passes=False, use_tc_tiling_on_sc=False),
        scratch_types=[
            pltpu.VMEM((CH,), jnp.int32),
            pltpu.VMEM((CH,), jnp.int32),
            pltpu.VMEM((CH,), jnp.int32),
            pltpu.VMEM((CH,), jnp.int32),
            pltpu.VMEM((CH,), jnp.int32),
            pltpu.VMEM((CH,), jnp.int32),
            pltpu.VMEM((CH, D), jnp.float32),
            pltpu.VMEM((CH, D), jnp.float32),
            pltpu.VMEM((ACC_ROWS, D), jnp.float32),
            pltpu.VMEM((16,), jnp.int32),
        ] + ([pltpu.VMEM_SHARED((NPAD, D), jnp.float32)] if use_spm else [])
        + [
            pltpu.SemaphoreType.DMA,
            pltpu.SemaphoreType.DMA,
            pltpu.SemaphoreType.DMA,
            pltpu.SemaphoreType.DMA,
        ],
    )


def _pool_body(x4, x1, x2, x3, batch_hbm, out_hbm, bv, rowbuf, acc, sem):
    wid = _wid()
    lo = wid * NLOC
    nrows = jnp.minimum(jnp.int32(NLOC), jnp.int32(N) - lo)
    ninf = jnp.full((16,), NEG_INF, jnp.float32)

    def ibody(r, _):
        for g in range(16):
            acc[r, pl.ds(g * 16, 16)] = ninf
        return 0

    lax.fori_loop(0, 64, ibody, 0)
    pltpu.sync_copy(batch_hbm.at[pl.ds(_mo8(lo), NLOC)], bv)
    for ai, arr in enumerate((x4, x1, x2, x3)):
        pltpu.sync_copy(arr.at[pl.ds(_mo8(lo), NLOC)], rowbuf)

        def rbody(j, _):
            bvec = bv[pl.ds(j * 16, 16)]
            for l in range(16):
                g = bvec[l]
                r = j * 16 + l
                for fg in range(4):
                    col = ai * 64 + fg * 16
                    cur = acc[g, pl.ds(col, 16)]
                    acc[g, pl.ds(col, 16)] = jnp.maximum(
                        cur, rowbuf[r, pl.ds(fg * 16, 16)])
            return 0

        lax.fori_loop(0, nrows // 16, rbody, 0)
    pltpu.sync_copy(acc, out_hbm.at[wid])


def _make_pool():
    return pl.kernel(
        _pool_body,
        out_type=jax.ShapeDtypeStruct((NW, 64, 256), jnp.float32),
        mesh=plsc.VectorSubcoreMesh(**_MESH),
        compiler_params=pltpu.CompilerParams(needs_layout_passes=False, use_tc_tiling_on_sc=False),
        scratch_types=[
            pltpu.VMEM((NLOC,), jnp.int32),
            pltpu.VMEM((NLOC, 64), jnp.float32),
            pltpu.VMEM((64, 256), jnp.float32),
            pltpu.SemaphoreType.DMA,
        ],
    )


_bucket = _make_bucket()
_prop64 = _make_prop(64)
_pool = _make_pool()

_BR = 256
_GRID = NPAD // _BR


def _blk(w):
    return pl.BlockSpec((_BR, w), lambda i: (i, 0))


def _full(shape):
    nd = len(shape)
    return pl.BlockSpec(shape, lambda i, _n=nd: (0,) * _n)


def _t0_body(deg_ref, x_ref, w_ref, dis_ref, a_ref, b2_ref, p_ref):
    deg = deg_ref[:, 0:1]
    dis = jnp.where(deg > 0, lax.rsqrt(jnp.maximum(deg, 1e-12)), 0.0)
    dis_ref[...] = jnp.broadcast_to(dis, (_BR, 64))
    x = x_ref[...]
    y0 = dis * x
    w = w_ref[...]
    a_ref[...] = jnp.dot(y0, w[1], preferred_element_type=jnp.float32)
    b2_ref[...] = jnp.dot(y0, w[2], preferred_element_type=jnp.float32)
    p_ref[...] = jnp.dot(x, w[0] - w[2], preferred_element_type=jnp.float32)


def _t0(deg16, xp, W):
    return pl.pallas_call(
        _t0_body,
        grid=(_GRID,),
        in_specs=[_blk(16), _blk(128), _full((3, 128, 64))],
        out_specs=[_blk(64)] * 4,
        out_shape=[jax.ShapeDtypeStruct((NPAD, 64), jnp.float32)] * 4,
    )(deg16, xp, W)


def _t2l1_body(p_ref, sa_ref, sc_ref, dis_ref, b_ref, out_ref, u_ref):
    dis = dis_ref[...]
    o = p_ref[...] - dis * sa_ref[...] - 2.0 * dis * sc_ref[...] + b_ref[...]
    o = jnp.where(o >= 0, o, ALPHA * o)
    out_ref[...] = o
    u_ref[...] = dis * o


def _t2l1(p, Sa, Sc, dis64, b):
    return pl.pallas_call(
        _t2l1_body,
        grid=(_GRID,),
        in_specs=[_blk(64)] * 4 + [_full((1, 64))],
        out_specs=[_blk(64)] * 2,
        out_shape=[jax.ShapeDtypeStruct((NPAD, 64), jnp.float32)] * 2,
    )(p, Sa, Sc, dis64, b)


def _t1_body(s1_ref, dis_ref, u_ref):
    dis = dis_ref[...]
    u_ref[...] = -(dis * dis) * s1_ref[...]


def _t1(S1, dis, D):
    return pl.pallas_call(
        _t1_body,
        grid=(_GRID,),
        in_specs=[_blk(D), _blk(D)],
        out_specs=_blk(D),
        out_shape=jax.ShapeDtypeStruct((NPAD, D), jnp.float32),
    )(S1, dis)


def _layer_math(h, S1, S2, dis, W, b):
    Tx1 = -dis * S1
    Tx2 = -2.0 * dis * S2 - h
    o = (jnp.dot(h, W[0], preferred_element_type=jnp.float32)
         + jnp.dot(Tx1, W[1], preferred_element_type=jnp.float32)
         + jnp.dot(Tx2, W[2], preferred_element_type=jnp.float32) + b)
    return jnp.where(o >= 0, o, ALPHA * o)


def _t2_body(emit_u, h_ref, s1_ref, s2_ref, dis_ref, w_ref, b_ref,
             out_ref, *rest):
    out = _layer_math(h_ref[...], s1_ref[...], s2_ref[...], dis_ref[...],
                      w_ref[...], b_ref[...])
    out_ref[...] = out
    if emit_u:
        rest[0][...] = dis_ref[:, 0:64] * out


def _t2(h, S1, S2, dis, W, b, D, emit_u):
    n_out = 2 if emit_u else 1
    outs = pl.pallas_call(
        functools.partial(_t2_body, emit_u),
        grid=(_GRID,),
        in_specs=[_blk(D), _blk(D), _blk(D), _blk(D),
                  _full((3, D, 64)), _full((1, 64))],
        out_specs=[_blk(64)] * n_out,
        out_shape=[jax.ShapeDtypeStruct((NPAD, 64), jnp.float32)] * n_out,
    )(h, S1, S2, dis, W, b)
    return outs if emit_u else outs[0]


def _t2_skip_body(h_ref, s1_ref, s2_ref, dis_ref, w_ref, b_ref, hs_ref,
                  x2_ref, x3_ref, u_ref):
    x2 = _layer_math(h_ref[...], s1_ref[...], s2_ref[...], dis_ref[...],
                     w_ref[...], b_ref[...])
    x3 = hs_ref[...] + x2
    x2_ref[...] = x2
    x3_ref[...] = x3
    u_ref[...] = dis_ref[...] * x3


def _t2_skip(h, S1, S2, dis, W, b, hskip):
    return pl.pallas_call(
        _t2_skip_body,
        grid=(_GRID,),
        in_specs=[_blk(64)] * 4 + [_full((3, 64, 64)), _full((1, 64)),
                                   _blk(64)],
        out_specs=[_blk(64)] * 3,
        out_shape=[jax.ShapeDtypeStruct((NPAD, 64), jnp.float32)] * 3,
    )(h, S1, S2, dis, W, b, hskip)


def _finish_body(part_ref, w1_ref, b1_ref, g_ref, be_ref, w2_ref, b2_ref,
                 out_ref):
    pooled = jnp.max(part_ref[...], axis=0)
    h = jnp.dot(pooled, w1_ref[...], preferred_element_type=jnp.float32)
    h = h + b1_ref[...]
    h = g_ref[...] * h * (1.0 / jnp.sqrt(1.0 + EPS)) + be_ref[...]
    h = jnp.maximum(h, 0.0)
    out_ref[...] = (jnp.dot(h, w2_ref[...],
                            preferred_element_type=jnp.float32) + b2_ref[...])


def _finish(part, W1, b1, gamma, beta, W2, b2):
    return pl.pallas_call(
        _finish_body,
        grid=(1,),
        in_specs=[_full((NW, 64, 256)), _full((256, 64)), _full((1, 64)),
                  _full((1, 64)), _full((1, 64)), _full((64, 10)),
                  _full((1, 10))],
        out_specs=_full((64, 10)),
        out_shape=jax.ShapeDtypeStruct((64, 10), jnp.float32),
    )(part, W1, b1, gamma, beta, W2, b2)


def kernel(x, edge_index, batch, W_init, b_init, W_head, b_head,
           W_body, b_body, W_tail, b_tail, W1, b1, gamma, beta, W2, b2):
    src = edge_index[0]
    dst = edge_index[1]
    xp = jnp.pad(x, ((0, NPAD - N), (0, 0)))
    batchp = jnp.pad(batch, (0, NPAD - N))

    packed, counts, deg_flat = _bucket(src, dst)
    deg16 = deg_flat.reshape(NPAD, 16)
    dis64, a_p, b_p, p_p = _t0(deg16, xp, W_init)

    def cheb(h, u_in, W, b, D, prop, emit_u):
        S1 = prop(u_in, packed, counts)
        u1 = _t1(S1, dis64, D)
        S2 = prop(u1, packed, counts)
        return _t2(h, S1, S2, dis64, W, b.reshape(1, 64), D, emit_u), S1, S2

    Sa = _prop64(a_p, packed, counts)
    Sb = _prop64(b_p, packed, counts)
    u_c = _t1(Sb, dis64, 64)
    Sc = _prop64(u_c, packed, counts)
    h0, u_h0 = _t2l1(p_p, Sa, Sc, dis64, b_init.reshape(1, 64))
    (x1, u_x1), _, _ = cheb(h0, u_h0, W_head, b_head, 64, _prop64, True)
    S1 = _prop64(u_x1, packed, counts)
    u1 = _t1(S1, dis64, 64)
    S2 = _prop64(u1, packed, counts)
    x2, x3, u_x3 = _t2_skip(x1, S1, S2, dis64, W_body,
                            b_body.reshape(1, 64), h0)
    x4, _, _ = cheb(x3, u_x3, W_tail, b_tail, 64, _prop64, False)

    part = _pool(x4, x1, x2, x3, batchp)
    return _finish(part, W1, b1.reshape(1, 64), gamma.reshape(1, 64),
                   beta.reshape(1, 64), W2, b2.reshape(1, 10))
